# baseline (speedup 1.0000x reference)
import dataclasses
import functools

import jax
import jax.numpy as jnp
import numpy as np
from jax import lax
from jax.experimental import pallas as pl
from jax.experimental.pallas import tpu as pltpu
from jax.experimental.pallas import tpu_sc as plsc

NDIM = 3
MAX_COLLISIONS = 128
L = 16
NSUB = 16
GMAX = 12
NCELL = GMAX * GMAX * GMAX
NCELL_PAD = 1792
BUFN = 160

_R2 = np.float32(0.01)
_TEN = np.float32(10.0)
_EPS = np.float32(-1e-6)


def _bf16_round(x):
    u = plsc.bitcast(x, jnp.uint32)
    r = u + np.uint32(0x7FFF) + ((u >> np.uint32(16)) & np.uint32(1))
    r = r & np.uint32(0xFFFF0000)
    return plsc.bitcast(r, jnp.float32)


def _iota16():
    return lax.iota(jnp.int32, L)


def _compiler_params():
    cp = pltpu.CompilerParams()
    if "needs_layout_passes" in pltpu.CompilerParams.__dataclass_fields__:
        cp = dataclasses.replace(cp, needs_layout_passes=False)
    return cp


def _st1(ref, idx, val):
    lane0 = _iota16() == 0
    plsc.store_scatter(ref, [jnp.full((L,), idx, jnp.int32)],
                       jnp.full((L,), val), mask=lane0)


def _rd(ref, idx):
    return ref[pl.ds(idx, L)][0]


def _phase1(locs_T, B):
    N = locs_T.shape[0] // (B * NDIM)
    CHUNK = N // NSUB
    NV = CHUNK // L
    QC = CHUNK // 4
    mesh = plsc.VectorSubcoreMesh(core_axis_name="c", subcore_axis_name="s")

    out_type = (
        jax.ShapeDtypeStruct((B * NDIM * N,), jnp.float32),
        jax.ShapeDtypeStruct((B * N,), jnp.float32),
        jax.ShapeDtypeStruct((B * N,), jnp.int32),
        jax.ShapeDtypeStruct((B * 4 * N,), jnp.float32),
        jax.ShapeDtypeStruct((B * NCELL_PAD,), jnp.int32),
        jax.ShapeDtypeStruct((B * 128,), jnp.float32),
        jax.ShapeDtypeStruct((B * 128,), jnp.int32),
    )
    scratch = [
        pltpu.VMEM((NDIM * CHUNK,), jnp.float32),
        pltpu.VMEM((4 * CHUNK,), jnp.float32),
        pltpu.VMEM((CHUNK,), jnp.int32),
        pltpu.VMEM((QC,), jnp.int32),
        pltpu.VMEM((QC,), jnp.int32),
        pltpu.VMEM((QC,), jnp.int32),
        pltpu.VMEM((QC,), jnp.int32),
        pltpu.VMEM((CHUNK,), jnp.float32),
        pltpu.VMEM((CHUNK,), jnp.int32),
        pltpu.VMEM((NCELL_PAD,), jnp.int32),
        pltpu.VMEM((L,), jnp.float32),
        pltpu.VMEM((NSUB * NCELL_PAD,), jnp.int32),
        pltpu.VMEM((NSUB * NCELL_PAD,), jnp.int32),
        pltpu.VMEM((NCELL_PAD,), jnp.int32),
        pltpu.VMEM((128,), jnp.float32),
        pltpu.VMEM((128,), jnp.int32),
        pltpu.VMEM((NSUB * L,), jnp.float32),
        pltpu.VMEM_SHARED((NSUB * L,), jnp.float32),
        pltpu.VMEM_SHARED((NSUB * NCELL_PAD,), jnp.int32),
        pltpu.VMEM_SHARED((NSUB * NCELL_PAD,), jnp.int32),
        pltpu.VMEM_SHARED((N,), jnp.float32),
        pltpu.VMEM_SHARED((N,), jnp.float32),
        pltpu.VMEM_SHARED((N,), jnp.float32),
        pltpu.VMEM_SHARED((N,), jnp.float32),
        pltpu.VMEM_SHARED((N,), jnp.float32),
        pltpu.VMEM_SHARED((N,), jnp.float32),
        pltpu.VMEM_SHARED((N,), jnp.float32),
        pltpu.VMEM_SHARED((N,), jnp.int32),
        pltpu.SemaphoreType.DMA,
    ]

    @functools.partial(pl.kernel, out_type=out_type, mesh=mesh,
                       scratch_types=scratch,
                       compiler_params=_compiler_params())
    def k(locs_hbm, locsr_hbm, idxs_hbm, permi_hbm, p_hbm,
          cs_hbm, mf_hbm, mi_hbm, xyz, bll, cid, pos0, pos1, pos2, pos3,
          orig, orig_i, hist, mmv, hall, oall, csl, mfs, mis, mm_all,
          sh_mm, sh_hist, sh_off, sh_xr, sh_yr, sh_zr, sh_bx, sh_by,
          sh_bz, sh_ll, sh_oi, sem):
        b = lax.axis_index("c")
        s = lax.axis_index("s")
        base = s * CHUNK
        it = _iota16()

        for a in range(NDIM):
            pltpu.sync_copy(
                locs_hbm.at[pl.ds(b * (NDIM * N) + a * N + base, CHUNK)],
                xyz.at[pl.ds(a * CHUNK, CHUNK)])

        inf = jnp.full((L,), jnp.inf, jnp.float32)
        vec = inf
        for a in range(NDIM):
            def red(i, carry):
                mn, nmx = carry
                v = xyz[pl.ds(a * CHUNK + i * L, L)]
                return jnp.minimum(mn, v), jnp.minimum(nmx, -v)
            mn, nmx = lax.fori_loop(0, NV, red, (inf, inf))
            vec = jnp.where(it == a, jnp.full((L,), jnp.min(mn)), vec)
            vec = jnp.where(it == NDIM + a,
                            jnp.full((L,), jnp.min(nmx)), vec)
        mmv[...] = vec
        pltpu.sync_copy(mmv, sh_mm.at[pl.ds(s * L, L)])
        plsc.subcore_barrier()

        pltpu.sync_copy(sh_mm, mm_all)
        acc = inf
        for t in range(NSUB):
            acc = jnp.minimum(acc, mm_all[pl.ds(t * L, L)])
        mmv[...] = acc
        i3 = jnp.minimum(it, 2)
        mn3 = plsc.load_gather(mmv, [i3])
        nmx3 = plsc.load_gather(mmv, [i3 + 3])
        lower = mn3 + _EPS
        delta = (np.float32(0.0) - nmx3) - lower
        t = delta * _TEN
        ti = lax.convert_element_type(t, jnp.int32)
        tif = lax.convert_element_type(ti, jnp.float32)
        one = jnp.full((L,), 1, jnp.int32)
        zero = jnp.zeros((L,), jnp.int32)
        g = jnp.minimum(ti + jnp.where(t > tif, one, zero) + 1, 96)
        lox = lower[0]
        loy = lower[1]
        loz = lower[2]
        gx = g[0]
        gy = g[1]
        gz = g[2]
        sy = gz
        sx = gy * gz

        gxv = jnp.full((L,), gx - 1, jnp.int32)
        gyv = jnp.full((L,), gy - 1, jnp.int32)
        gzv = jnp.full((L,), gz - 1, jnp.int32)
        sxv = jnp.full((L,), sx, jnp.int32)
        syv = jnp.full((L,), sy, jnp.int32)

        def cells(i, _):
            sl = pl.ds(i * L, L)
            x = xyz[pl.ds(0 * CHUNK + i * L, L)]
            y = xyz[pl.ds(1 * CHUNK + i * L, L)]
            z = xyz[pl.ds(2 * CHUNK + i * L, L)]
            cx = lax.convert_element_type((x - lox) * _TEN, jnp.int32)
            cy = lax.convert_element_type((y - loy) * _TEN, jnp.int32)
            cz = lax.convert_element_type((z - loz) * _TEN, jnp.int32)
            cx = jnp.minimum(jnp.maximum(cx, zero), gxv)
            cy = jnp.minimum(jnp.maximum(cy, zero), gyv)
            cz = jnp.minimum(jnp.maximum(cz, zero), gzv)
            cid[sl] = cx * sxv + cy * syv + cz
            bll[pl.ds(0 * CHUNK + i * L, L)] = _bf16_round(x)
            bll[pl.ds(1 * CHUNK + i * L, L)] = _bf16_round(y)
            bll[pl.ds(2 * CHUNK + i * L, L)] = _bf16_round(z)
            bll[pl.ds(3 * CHUNK + i * L, L)] = (x * x + y * y) + z * z
            oi = jnp.full((L,), base + i * L, jnp.int32) + it
            orig_i[sl] = oi
            orig[sl] = lax.convert_element_type(oi, jnp.float32)
            return 0
        lax.fori_loop(0, NV, cells, 0)

        @pl.loop(0, NCELL_PAD, step=L)
        def _(i):
            hist[pl.ds(i, L)] = zero

        @pl.loop(0, CHUNK, step=L)
        def _(i):
            cv = cid[pl.ds(i, L)]
            for kk in range(L):
                c = cv[kk]
                _st1(hist, c, _rd(hist, c) + 1)

        pltpu.sync_copy(hist, sh_hist.at[pl.ds(s * NCELL_PAD, NCELL_PAD)])
        plsc.subcore_barrier()

        @pl.when(s == 0)
        def _():
            pltpu.sync_copy(sh_hist, hall)

            @pl.loop(NCELL, NCELL_PAD, step=L)
            def _(i):
                csl[pl.ds(i, L)] = jnp.full((L,), N, jnp.int32)

            def scan(kk, bb):
                kv = it * NCELL_PAD + kk
                cnt = plsc.load_gather(hall, [kv])
                incl = plsc.cumsum(cnt)
                excl = incl - cnt
                plsc.store_scatter(oall, [kv],
                                   jnp.full((L,), bb, jnp.int32) + excl)
                _st1(csl, kk, bb)
                return bb + jnp.sum(cnt)
            lax.fori_loop(0, NCELL, scan, np.int32(0))

            pltpu.sync_copy(oall, sh_off)
            pltpu.sync_copy(csl, cs_hbm.at[pl.ds(b * NCELL_PAD, NCELL_PAD)])
            mfs[pl.ds(0, L)] = lower
            mis[pl.ds(0, L)] = jnp.where(
                it == 3, jnp.full((L,), sx, jnp.int32),
                jnp.where(it == 4, jnp.full((L,), sy, jnp.int32), g))
            pltpu.sync_copy(mfs, mf_hbm.at[pl.ds(b * 128, 128)])
            pltpu.sync_copy(mis, mi_hbm.at[pl.ds(b * 128, 128)])
        plsc.subcore_barrier()

        pltpu.sync_copy(sh_off.at[pl.ds(s * NCELL_PAD, NCELL_PAD)], hist)

        for ch, pref in enumerate((pos0, pos1, pos2, pos3)):
            @pl.loop(ch * QC, (ch + 1) * QC, step=L)
            def _(i, pref=pref, ch=ch):
                cv = cid[pl.ds(i, L)]
                for kk in range(L):
                    c = cv[kk]
                    p = _rd(hist, c)
                    _st1(hist, c, p + 1)
                    _st1(pref, i + kk - ch * QC, p)

        copies = []
        for ch, pref in enumerate((pos0, pos1, pos2, pos3)):
            qs = pl.ds(ch * QC, QC)
            copies.append(pltpu.async_copy(
                orig_i.at[qs], sh_oi.at[pref], sem))
            for a, shr in enumerate((sh_xr, sh_yr, sh_zr)):
                copies.append(pltpu.async_copy(
                    xyz.at[pl.ds(a * CHUNK + ch * QC, QC)],
                    shr.at[pref], sem))
            for a, shr in enumerate((sh_bx, sh_by, sh_bz, sh_ll)):
                copies.append(pltpu.async_copy(
                    bll.at[pl.ds(a * CHUNK + ch * QC, QC)],
                    shr.at[pref], sem))
        for cp in copies:
            cp.wait()
        plsc.subcore_barrier()

        csl2 = pl.ds(base, CHUNK)
        for a, shr in enumerate((sh_xr, sh_yr, sh_zr)):
            pltpu.sync_copy(
                shr.at[csl2],
                locsr_hbm.at[pl.ds(b * (NDIM * N) + a * N + base, CHUNK)])
        for a, shr in enumerate((sh_bx, sh_by, sh_bz, sh_ll)):
            pltpu.sync_copy(
                shr.at[csl2],
                p_hbm.at[pl.ds(b * (4 * N) + a * N + base, CHUNK)])
        pltpu.sync_copy(sh_oi.at[csl2],
                        permi_hbm.at[pl.ds(b * N + base, CHUNK)])
        pltpu.sync_copy(sh_oi.at[csl2], orig_i)

        @pl.loop(0, NV, step=1)
        def _(i):
            orig[pl.ds(i * L, L)] = lax.convert_element_type(
                orig_i[pl.ds(i * L, L)], jnp.float32)
        pltpu.sync_copy(orig, idxs_hbm.at[pl.ds(b * N + base, CHUNK)])

    return k(locs_T)


def _phase2(qlocs_T, data2, permi, p_arr, cs, mf, mi, C):
    B = data2.shape[0]
    M = qlocs_T.shape[0] // (B * NDIM)
    N = p_arr.shape[0] // (B * 4)
    NP = N + L
    CHUNK = M // NSUB
    CP = CHUNK + L
    DC = N // NSUB
    QC = DC // 4
    QG = 32
    mesh = plsc.VectorSubcoreMesh(core_axis_name="c", subcore_axis_name="s")

    out_type = (
        jax.ShapeDtypeStruct((B * M * MAX_COLLISIONS,), jnp.float32),
        jax.ShapeDtypeStruct((B * N * C,), jnp.float32),
    )
    scratch = [
        pltpu.VMEM((4 * NP,), jnp.float32),
        pltpu.VMEM((NCELL_PAD,), jnp.int32),
        pltpu.VMEM((NDIM * CHUNK,), jnp.float32),
        pltpu.VMEM((4 * CP,), jnp.float32),
        pltpu.VMEM((NDIM * CP,), jnp.int32),
        pltpu.VMEM((NDIM * CP,), jnp.float32),
        pltpu.VMEM((6 * L,), jnp.int32),
        pltpu.VMEM((BUFN,), jnp.float32),
        pltpu.VMEM((QG * MAX_COLLISIONS,), jnp.float32),
        pltpu.VMEM((128,), jnp.float32),
        pltpu.VMEM((128,), jnp.int32),
        pltpu.VMEM((QC,), jnp.int32),
        pltpu.VMEM((QC,), jnp.int32),
        pltpu.VMEM((QC,), jnp.int32),
        pltpu.VMEM((QC,), jnp.int32),
        pltpu.VMEM((DC + L,), jnp.int32),
        pltpu.VMEM((DC, 128), jnp.float32),
        pltpu.VMEM((128 * 64,), jnp.float32),
        pltpu.SemaphoreType.DMA,
        pltpu.SemaphoreType.DMA,
    ]

    @functools.partial(pl.kernel, out_type=out_type, mesh=mesh,
                       scratch_types=scratch,
                       compiler_params=_compiler_params())
    def k(q_hbm, data_hbm, permi_hbm, p_hbm, cs_hbm, mf_hbm, mi_hbm,
          nb_hbm, datar_hbm, pv, csv, qv, bq, cq, uq, rows_se,
          buf, stg, mfs, mis, gi0, gi1, gi2, gi3, permL, drows, bounce,
          sem, gsem):
        b = lax.axis_index("c")
        s = lax.axis_index("s")
        qbase = s * CHUNK
        dbase = s * DC
        it = _iota16()

        pltpu.sync_copy(permi_hbm.at[pl.ds(b * N + dbase, DC)],
                        permL.at[pl.ds(0, DC)])
        girefs = (gi0, gi1, gi2, gi3)
        for ch, gir in enumerate(girefs):
            @pl.loop(0, QC, step=L)
            def _(i, gir=gir, ch=ch):
                gir[pl.ds(i, L)] = permL[pl.ds(ch * QC + i, L)] >> 1
        gathers = [
            pltpu.async_copy(data_hbm.at[b].at[gir],
                             drows.at[pl.ds(ch * QC, QC), :], gsem)
            for ch, gir in enumerate(girefs)
        ]

        for a in range(4):
            pltpu.sync_copy(p_hbm.at[pl.ds(b * (4 * N) + a * N, N)],
                            pv.at[pl.ds(a * NP, N)])
        pltpu.sync_copy(cs_hbm.at[pl.ds(b * NCELL_PAD, NCELL_PAD)], csv)
        pltpu.sync_copy(mf_hbm.at[pl.ds(b * 128, 128)], mfs)
        pltpu.sync_copy(mi_hbm.at[pl.ds(b * 128, 128)], mis)
        for a in range(NDIM):
            pltpu.sync_copy(
                q_hbm.at[pl.ds(b * (NDIM * M) + a * M + qbase, CHUNK)],
                qv.at[pl.ds(a * CHUNK, CHUNK)])

        mv = mfs[pl.ds(0, L)]
        miv = mis[pl.ds(0, L)]
        lox = mv[0]
        loy = mv[1]
        loz = mv[2]
        gx = miv[0]
        gy = miv[1]
        gz = miv[2]
        sx = miv[3]
        sy = miv[4]

        one = jnp.full((L,), 1, jnp.int32)
        zero = jnp.zeros((L,), jnp.int32)

        def prep(i, _):
            x = qv[pl.ds(0 * CHUNK + i * L, L)]
            y = qv[pl.ds(1 * CHUNK + i * L, L)]
            z = qv[pl.ds(2 * CHUNK + i * L, L)]
            bq[pl.ds(0 * CP + i * L, L)] = _bf16_round(x)
            bq[pl.ds(1 * CP + i * L, L)] = _bf16_round(y)
            bq[pl.ds(2 * CP + i * L, L)] = _bf16_round(z)
            bq[pl.ds(3 * CP + i * L, L)] = (x * x + y * y) + z * z
            for a, w in ((0, x - lox), (1, y - loy), (2, z - loz)):
                t = w * _TEN
                ti = lax.convert_element_type(t, jnp.int32)
                tif = lax.convert_element_type(ti, jnp.float32)
                cq[pl.ds(a * CP + i * L, L)] = ti - jnp.where(t < tif, one,
                                                              zero)
                uq[pl.ds(a * CP + i * L, L)] = t
            return 0
        lax.fori_loop(0, CHUNK // L, prep, 0)

        neg1 = jnp.full((L,), -1.0, jnp.float32)
        r2v = jnp.full((L,), _R2, jnp.float32)

        def do_query(q):
            i3m = jnp.minimum(it, 2)
            qfv = jnp.full((L,), q, jnp.int32)
            qb4 = plsc.load_gather(bq, [qfv + CP * jnp.minimum(it, 3)])
            cq3 = plsc.load_gather(cq, [qfv + CP * i3m])
            uq3 = plsc.load_gather(uq, [qfv + CP * i3m])
            bqx = jnp.full((L,), qb4[0], jnp.float32)
            bqy = jnp.full((L,), qb4[1], jnp.float32)
            bqz = jnp.full((L,), qb4[2], jnp.float32)
            qqv = jnp.full((L,), qb4[3], jnp.float32)
            cqx = cq3[0]
            cqy = cq3[1]
            cqz = cq3[2]

            @pl.loop(0, BUFN, step=L)
            def _(i):
                buf[pl.ds(i, L)] = neg1

            tc2 = jnp.full((L,), np.float32(3.38), jnp.float32)
            uxv = jnp.full((L,), uq3[0], jnp.float32)
            uyv = jnp.full((L,), uq3[1], jnp.float32)
            uzv = jnp.full((L,), uq3[2], jnp.float32)
            czv = jnp.full((L,), cqz, jnp.int32)
            zoks = jnp.logical_and(cqz + 2 >= 0, cqz - 2 <= gz - 1)
            fone = jnp.full((L,), np.float32(1.0), jnp.float32)
            fzero = jnp.zeros((L,), jnp.float32)

            for h in range(2):
                rr = it + h * L
                rx = rr // 5 - 2
                ry = rr % 5 - 2
                cx = jnp.full((L,), cqx, jnp.int32) + rx
                cy = jnp.full((L,), cqy, jnp.int32) + ry
                cxf = lax.convert_element_type(cx, jnp.float32)
                cyf = lax.convert_element_type(cy, jnp.float32)
                dxm = jnp.maximum(
                    jnp.maximum(cxf - uxv, uxv - cxf - fone), fzero)
                dym = jnp.maximum(
                    jnp.maximum(cyf - uyv, uyv - cyf - fone), fzero)
                rxy2 = dxm * dxm + dym * dym
                valid = jnp.logical_and(
                    jnp.logical_and(rr < 25, rxy2 <= tc2),
                    jnp.logical_and(
                        jnp.logical_and(cx >= 0, cx < jnp.full((L,), gx,
                                                            jnp.int32)),
                        jnp.logical_and(cy >= 0, cy < jnp.full((L,), gy,
                                                            jnp.int32))))
                valid = jnp.logical_and(valid, zoks)
                thr = tc2 - rxy2

                def zfail(zc):
                    czf = lax.convert_element_type(zc, jnp.float32)
                    dz = jnp.maximum(
                        jnp.maximum(czf - uzv, uzv - czf - fone), fzero)
                    return lax.convert_element_type(dz * dz > thr,
                                                    jnp.int32)

                zlo_r = jnp.maximum(czv - 2 + zfail(czv - 2)
                                    + zfail(czv - 1), zero)
                zhi_r = jnp.minimum(czv + 2 - zfail(czv + 2)
                                    - zfail(czv + 1),
                                    jnp.full((L,), gz - 1, jnp.int32))
                cxc = jnp.minimum(jnp.maximum(cx, zero),
                                  jnp.full((L,), gx - 1, jnp.int32))
                cyc = jnp.minimum(jnp.maximum(cy, zero),
                                  jnp.full((L,), gy - 1, jnp.int32))
                bc = cxc * jnp.full((L,), sx, jnp.int32) \
                    + cyc * jnp.full((L,), sy, jnp.int32)
                valid = jnp.logical_and(valid, zhi_r >= zlo_r)
                sidx = bc + jnp.maximum(zlo_r, zero)
                eidx = bc + jnp.minimum(zhi_r,
                                        jnp.full((L,), GMAX - 1,
                                                 jnp.int32)) + 1
                s0v = plsc.load_gather(csv, [sidx])
                e0v = plsc.load_gather(csv, [eidx])
                e0v = jnp.where(valid, e0v, s0v)
                ii = jnp.full((L,), 2 * h * L, jnp.int32) + 2 * it
                plsc.store_scatter(rows_se, [ii], s0v)
                plsc.store_scatter(rows_se, [ii + 1], e0v)

            bufcap = jnp.full((L,), BUFN, jnp.int32)

            def row(r, cnt):
                v = rows_se[pl.ds(2 * r, L)]
                s0 = v[0]
                e = v[1]

                def cond(carry):
                    return carry[0] < e

                def body(carry):
                    j, cv = carry
                    rem = e - j
                    lmask = it < jnp.full((L,), rem, jnp.int32)
                    xv = pv[pl.ds(0 * NP + j, L)]
                    yv = pv[pl.ds(1 * NP + j, L)]
                    zv = pv[pl.ds(2 * NP + j, L)]
                    llv = pv[pl.ds(3 * NP + j, L)]
                    mm = (bqx * xv + bqy * yv) + bqz * zv
                    d2 = (qqv + llv) - (np.float32(2.0) * mm)
                    hit = jnp.logical_and(d2 <= r2v, lmask)
                    hi = lax.convert_element_type(hit, jnp.int32)
                    idxv = cv + plsc.cumsum(hi) - hi
                    posv = lax.convert_element_type(
                        jnp.full((L,), j, jnp.int32) + it, jnp.float32)
                    plsc.store_scatter(
                        buf, [idxv], posv,
                        mask=jnp.logical_and(hit, idxv < bufcap))
                    return j + L, cv + plsc.all_reduce_population_count(hit)
                _, cnt = lax.while_loop(cond, body, (s0, cnt))
                return cnt
            lax.fori_loop(0, 25, row, jnp.zeros((L,), jnp.int32))

        def group(qc, _):
            def one_q(qs, _):
                q = qc * QG + qs
                do_query(q)

                @pl.loop(0, MAX_COLLISIONS, step=L)
                def _(i):
                    stg[pl.ds(qs * MAX_COLLISIONS + i, L)] = buf[pl.ds(i, L)]
                return 0
            lax.fori_loop(0, QG, one_q, 0)
            pltpu.sync_copy(
                stg,
                nb_hbm.at[pl.ds(
                    b * (M * MAX_COLLISIONS)
                    + (qbase + qc * QG) * MAX_COLLISIONS,
                    QG * MAX_COLLISIONS)])
            return 0
        lax.fori_loop(0, CHUNK // QG, group, 0)

        for gcp in gathers:
            gcp.wait()

        for ch2 in range(4):
            r0 = ch2 * (DC // 4)

            @pl.loop(r0, r0 + DC // 4)
            def _(i, r0=r0):
                h = (_rd(permL, i) & 1) * C
                for kq in range(C // L):
                    v = plsc.load_gather(
                        drows, [jnp.full((L,), i, jnp.int32),
                                jnp.full((L,), h + kq * L, jnp.int32) + it])
                    bounce[pl.ds((i - r0) * C + kq * L, L)] = v

            pltpu.sync_copy(
                bounce,
                datar_hbm.at[pl.ds(b * (N * C) + (dbase + r0) * C,
                                   (DC // 4) * C)])

    return k(qlocs_T, data2, permi, p_arr, cs, mf, mi)


def kernel(locs, data, qlocs):
    B, N, _ = locs.shape
    M = qlocs.shape[1]
    locs_T = jnp.transpose(locs, (0, 2, 1)).reshape(B * NDIM * N)
    qlocs_T = jnp.transpose(qlocs, (0, 2, 1)).reshape(B * NDIM * M)
    C = data.shape[2]
    data2 = data.reshape(B, (N * C) // 128, 128)
    locsr_T, idxs, permi, p_arr, cs, mf, mi = _phase1(locs_T, B)
    neighbors, data_r = _phase2(qlocs_T, data2, permi, p_arr, cs, mf, mi, C)
    locs_r = jnp.transpose(locsr_T.reshape(B, NDIM, N), (0, 2, 1))
    return (locs_r, data_r.reshape(B, N, C), idxs.reshape(B, N),
            neighbors.reshape(B, M, MAX_COLLISIONS))


# speedup vs baseline: 1.0533x; 1.0533x over previous
import dataclasses
import functools

import jax
import jax.numpy as jnp
import numpy as np
from jax import lax
from jax.experimental import pallas as pl
from jax.experimental.pallas import tpu as pltpu
from jax.experimental.pallas import tpu_sc as plsc

NDIM = 3
MAX_COLLISIONS = 128
L = 16
NSUB = 16
GMAX = 12
NCELL = GMAX * GMAX * GMAX
NCELL_PAD = 1792
BUFN = 160

_R2 = np.float32(0.01)
_TEN = np.float32(10.0)
_EPS = np.float32(-1e-6)


def _bf16_round(x):
    u = plsc.bitcast(x, jnp.uint32)
    r = u + np.uint32(0x7FFF) + ((u >> np.uint32(16)) & np.uint32(1))
    r = r & np.uint32(0xFFFF0000)
    return plsc.bitcast(r, jnp.float32)


def _iota16():
    return lax.iota(jnp.int32, L)


def _compiler_params():
    cp = pltpu.CompilerParams()
    if "needs_layout_passes" in pltpu.CompilerParams.__dataclass_fields__:
        cp = dataclasses.replace(cp, needs_layout_passes=False)
    return cp


def _st1(ref, idx, val):
    lane0 = _iota16() == 0
    plsc.store_scatter(ref, [jnp.full((L,), idx, jnp.int32)],
                       jnp.full((L,), val), mask=lane0)


def _rd(ref, idx):
    return ref[pl.ds(idx, L)][0]


def _phase1(locs_T, B):
    N = locs_T.shape[0] // (B * NDIM)
    CHUNK = N // NSUB
    NV = CHUNK // L
    QC = CHUNK // 4
    mesh = plsc.VectorSubcoreMesh(core_axis_name="c", subcore_axis_name="s")

    out_type = (
        jax.ShapeDtypeStruct((B * NDIM * N,), jnp.float32),
        jax.ShapeDtypeStruct((B * N,), jnp.float32),
        jax.ShapeDtypeStruct((B * N,), jnp.int32),
        jax.ShapeDtypeStruct((B * 4 * N,), jnp.float32),
        jax.ShapeDtypeStruct((B * NCELL_PAD,), jnp.int32),
        jax.ShapeDtypeStruct((B * 128,), jnp.float32),
        jax.ShapeDtypeStruct((B * 128,), jnp.int32),
    )
    scratch = [
        pltpu.VMEM((NDIM * CHUNK,), jnp.float32),
        pltpu.VMEM((4 * CHUNK,), jnp.float32),
        pltpu.VMEM((CHUNK,), jnp.int32),
        pltpu.VMEM((QC,), jnp.int32),
        pltpu.VMEM((QC,), jnp.int32),
        pltpu.VMEM((QC,), jnp.int32),
        pltpu.VMEM((QC,), jnp.int32),
        pltpu.VMEM((CHUNK,), jnp.float32),
        pltpu.VMEM((CHUNK,), jnp.int32),
        pltpu.VMEM((NCELL_PAD,), jnp.int32),
        pltpu.VMEM((L,), jnp.float32),
        pltpu.VMEM((NSUB * NCELL_PAD,), jnp.int32),
        pltpu.VMEM((NSUB * NCELL_PAD,), jnp.int32),
        pltpu.VMEM((NCELL_PAD,), jnp.int32),
        pltpu.VMEM((128,), jnp.float32),
        pltpu.VMEM((128,), jnp.int32),
        pltpu.VMEM((NSUB * L,), jnp.float32),
        pltpu.VMEM_SHARED((NSUB * L,), jnp.float32),
        pltpu.VMEM_SHARED((NSUB * NCELL_PAD,), jnp.int32),
        pltpu.VMEM_SHARED((NSUB * NCELL_PAD,), jnp.int32),
        pltpu.VMEM_SHARED((N,), jnp.float32),
        pltpu.VMEM_SHARED((N,), jnp.float32),
        pltpu.VMEM_SHARED((N,), jnp.float32),
        pltpu.VMEM_SHARED((N,), jnp.float32),
        pltpu.VMEM_SHARED((N,), jnp.float32),
        pltpu.VMEM_SHARED((N,), jnp.float32),
        pltpu.VMEM_SHARED((N,), jnp.float32),
        pltpu.VMEM_SHARED((N,), jnp.int32),
        pltpu.SemaphoreType.DMA,
    ]

    @functools.partial(pl.kernel, out_type=out_type, mesh=mesh,
                       scratch_types=scratch,
                       compiler_params=_compiler_params())
    def k(locs_hbm, locsr_hbm, idxs_hbm, permi_hbm, p_hbm,
          cs_hbm, mf_hbm, mi_hbm, xyz, bll, cid, pos0, pos1, pos2, pos3,
          orig, orig_i, hist, mmv, hall, oall, csl, mfs, mis, mm_all,
          sh_mm, sh_hist, sh_off, sh_xr, sh_yr, sh_zr, sh_bx, sh_by,
          sh_bz, sh_ll, sh_oi, sem):
        b = lax.axis_index("c")
        s = lax.axis_index("s")
        base = s * CHUNK
        it = _iota16()

        for a in range(NDIM):
            pltpu.sync_copy(
                locs_hbm.at[pl.ds(b * (NDIM * N) + a * N + base, CHUNK)],
                xyz.at[pl.ds(a * CHUNK, CHUNK)])

        inf = jnp.full((L,), jnp.inf, jnp.float32)
        vec = inf
        for a in range(NDIM):
            def red(i, carry):
                mn, nmx = carry
                v = xyz[pl.ds(a * CHUNK + i * L, L)]
                return jnp.minimum(mn, v), jnp.minimum(nmx, -v)
            mn, nmx = lax.fori_loop(0, NV, red, (inf, inf))
            vec = jnp.where(it == a, jnp.full((L,), jnp.min(mn)), vec)
            vec = jnp.where(it == NDIM + a,
                            jnp.full((L,), jnp.min(nmx)), vec)
        mmv[...] = vec
        pltpu.sync_copy(mmv, sh_mm.at[pl.ds(s * L, L)])
        plsc.subcore_barrier()

        pltpu.sync_copy(sh_mm, mm_all)
        acc = inf
        for t in range(NSUB):
            acc = jnp.minimum(acc, mm_all[pl.ds(t * L, L)])
        mmv[...] = acc
        i3 = jnp.minimum(it, 2)
        mn3 = plsc.load_gather(mmv, [i3])
        nmx3 = plsc.load_gather(mmv, [i3 + 3])
        lower = mn3 + _EPS
        delta = (np.float32(0.0) - nmx3) - lower
        t = delta * _TEN
        ti = lax.convert_element_type(t, jnp.int32)
        tif = lax.convert_element_type(ti, jnp.float32)
        one = jnp.full((L,), 1, jnp.int32)
        zero = jnp.zeros((L,), jnp.int32)
        g = jnp.minimum(ti + jnp.where(t > tif, one, zero) + 1, 96)
        lox = lower[0]
        loy = lower[1]
        loz = lower[2]
        gx = g[0]
        gy = g[1]
        gz = g[2]
        sy = gz
        sx = gy * gz

        gxv = jnp.full((L,), gx - 1, jnp.int32)
        gyv = jnp.full((L,), gy - 1, jnp.int32)
        gzv = jnp.full((L,), gz - 1, jnp.int32)
        sxv = jnp.full((L,), sx, jnp.int32)
        syv = jnp.full((L,), sy, jnp.int32)

        def cells(i, _):
            sl = pl.ds(i * L, L)
            x = xyz[pl.ds(0 * CHUNK + i * L, L)]
            y = xyz[pl.ds(1 * CHUNK + i * L, L)]
            z = xyz[pl.ds(2 * CHUNK + i * L, L)]
            cx = lax.convert_element_type((x - lox) * _TEN, jnp.int32)
            cy = lax.convert_element_type((y - loy) * _TEN, jnp.int32)
            cz = lax.convert_element_type((z - loz) * _TEN, jnp.int32)
            cx = jnp.minimum(jnp.maximum(cx, zero), gxv)
            cy = jnp.minimum(jnp.maximum(cy, zero), gyv)
            cz = jnp.minimum(jnp.maximum(cz, zero), gzv)
            cid[sl] = cx * sxv + cy * syv + cz
            bll[pl.ds(0 * CHUNK + i * L, L)] = _bf16_round(x)
            bll[pl.ds(1 * CHUNK + i * L, L)] = _bf16_round(y)
            bll[pl.ds(2 * CHUNK + i * L, L)] = _bf16_round(z)
            bll[pl.ds(3 * CHUNK + i * L, L)] = (x * x + y * y) + z * z
            oi = jnp.full((L,), base + i * L, jnp.int32) + it
            orig_i[sl] = oi
            orig[sl] = lax.convert_element_type(oi, jnp.float32)
            return 0
        lax.fori_loop(0, NV, cells, 0)

        @pl.loop(0, NCELL_PAD, step=L)
        def _(i):
            hist[pl.ds(i, L)] = zero

        @pl.loop(0, CHUNK, step=L)
        def _(i):
            cv = cid[pl.ds(i, L)]
            for kk in range(L):
                c = cv[kk]
                _st1(hist, c, _rd(hist, c) + 1)

        pltpu.sync_copy(hist, sh_hist.at[pl.ds(s * NCELL_PAD, NCELL_PAD)])
        plsc.subcore_barrier()

        @pl.when(s == 0)
        def _():
            pltpu.sync_copy(sh_hist, hall)

            @pl.loop(NCELL, NCELL_PAD, step=L)
            def _(i):
                csl[pl.ds(i, L)] = jnp.full((L,), N, jnp.int32)

            def scan(kk, bb):
                kv = it * NCELL_PAD + kk
                cnt = plsc.load_gather(hall, [kv])
                incl = plsc.cumsum(cnt)
                excl = incl - cnt
                plsc.store_scatter(oall, [kv],
                                   jnp.full((L,), bb, jnp.int32) + excl)
                _st1(csl, kk, bb)
                return bb + jnp.sum(cnt)
            lax.fori_loop(0, NCELL, scan, np.int32(0))

            pltpu.sync_copy(oall, sh_off)
            pltpu.sync_copy(csl, cs_hbm.at[pl.ds(b * NCELL_PAD, NCELL_PAD)])
            mfs[pl.ds(0, L)] = lower
            mis[pl.ds(0, L)] = jnp.where(
                it == 3, jnp.full((L,), sx, jnp.int32),
                jnp.where(it == 4, jnp.full((L,), sy, jnp.int32), g))
            pltpu.sync_copy(mfs, mf_hbm.at[pl.ds(b * 128, 128)])
            pltpu.sync_copy(mis, mi_hbm.at[pl.ds(b * 128, 128)])
        plsc.subcore_barrier()

        pltpu.sync_copy(sh_off.at[pl.ds(s * NCELL_PAD, NCELL_PAD)], hist)

        for ch, pref in enumerate((pos0, pos1, pos2, pos3)):
            @pl.loop(ch * QC, (ch + 1) * QC, step=L)
            def _(i, pref=pref, ch=ch):
                cv = cid[pl.ds(i, L)]
                for kk in range(L):
                    c = cv[kk]
                    p = _rd(hist, c)
                    _st1(hist, c, p + 1)
                    _st1(pref, i + kk - ch * QC, p)

        copies = []
        for ch, pref in enumerate((pos0, pos1, pos2, pos3)):
            qs = pl.ds(ch * QC, QC)
            copies.append(pltpu.async_copy(
                orig_i.at[qs], sh_oi.at[pref], sem))
            for a, shr in enumerate((sh_xr, sh_yr, sh_zr)):
                copies.append(pltpu.async_copy(
                    xyz.at[pl.ds(a * CHUNK + ch * QC, QC)],
                    shr.at[pref], sem))
            for a, shr in enumerate((sh_bx, sh_by, sh_bz, sh_ll)):
                copies.append(pltpu.async_copy(
                    bll.at[pl.ds(a * CHUNK + ch * QC, QC)],
                    shr.at[pref], sem))
        for cp in copies:
            cp.wait()
        plsc.subcore_barrier()

        csl2 = pl.ds(base, CHUNK)
        for a, shr in enumerate((sh_xr, sh_yr, sh_zr)):
            pltpu.sync_copy(
                shr.at[csl2],
                locsr_hbm.at[pl.ds(b * (NDIM * N) + a * N + base, CHUNK)])
        for a, shr in enumerate((sh_bx, sh_by, sh_bz, sh_ll)):
            pltpu.sync_copy(
                shr.at[csl2],
                p_hbm.at[pl.ds(b * (4 * N) + a * N + base, CHUNK)])
        pltpu.sync_copy(sh_oi.at[csl2],
                        permi_hbm.at[pl.ds(b * N + base, CHUNK)])
        pltpu.sync_copy(sh_oi.at[csl2], orig_i)

        @pl.loop(0, NV, step=1)
        def _(i):
            orig[pl.ds(i * L, L)] = lax.convert_element_type(
                orig_i[pl.ds(i * L, L)], jnp.float32)
        pltpu.sync_copy(orig, idxs_hbm.at[pl.ds(b * N + base, CHUNK)])

    return k(locs_T)


def _phase2(qlocs_T, data2, permi, p_arr, cs, mf, mi, C):
    B = data2.shape[0]
    M = qlocs_T.shape[0] // (B * NDIM)
    N = p_arr.shape[0] // (B * 4)
    NP = N + L
    CHUNK = M // NSUB
    CP = CHUNK + L
    DC = N // NSUB
    QC = DC // 4
    QG = 32
    mesh = plsc.VectorSubcoreMesh(core_axis_name="c", subcore_axis_name="s")

    out_type = (
        jax.ShapeDtypeStruct((B * M * MAX_COLLISIONS,), jnp.float32),
        jax.ShapeDtypeStruct((B * N * C,), jnp.float32),
    )
    scratch = [
        pltpu.VMEM((4 * NP,), jnp.float32),
        pltpu.VMEM((NCELL_PAD,), jnp.int32),
        pltpu.VMEM((NDIM * CHUNK,), jnp.float32),
        pltpu.VMEM((4 * CP,), jnp.float32),
        pltpu.VMEM((NDIM * CP,), jnp.int32),
        pltpu.VMEM((NDIM * CP,), jnp.float32),
        pltpu.VMEM((6 * L,), jnp.int32),
        pltpu.VMEM((BUFN,), jnp.float32),
        pltpu.VMEM((QG * MAX_COLLISIONS,), jnp.float32),
        pltpu.VMEM((128,), jnp.float32),
        pltpu.VMEM((128,), jnp.int32),
        pltpu.VMEM((QC,), jnp.int32),
        pltpu.VMEM((QC,), jnp.int32),
        pltpu.VMEM((QC,), jnp.int32),
        pltpu.VMEM((QC,), jnp.int32),
        pltpu.VMEM((DC + L,), jnp.int32),
        pltpu.VMEM((DC, 128), jnp.float32),
        pltpu.VMEM((128 * 64,), jnp.float32),
        pltpu.SemaphoreType.DMA,
        pltpu.SemaphoreType.DMA,
    ]

    @functools.partial(pl.kernel, out_type=out_type, mesh=mesh,
                       scratch_types=scratch,
                       compiler_params=_compiler_params())
    def k(q_hbm, data_hbm, permi_hbm, p_hbm, cs_hbm, mf_hbm, mi_hbm,
          nb_hbm, datar_hbm, pv, csv, qv, bq, cq, uq, rows_se,
          buf, stg, mfs, mis, gi0, gi1, gi2, gi3, permL, drows, bounce,
          sem, gsem):
        b = lax.axis_index("c")
        s = lax.axis_index("s")
        qbase = s * CHUNK
        dbase = s * DC
        it = _iota16()

        pltpu.sync_copy(permi_hbm.at[pl.ds(b * N + dbase, DC)],
                        permL.at[pl.ds(0, DC)])
        girefs = (gi0, gi1, gi2, gi3)
        for ch, gir in enumerate(girefs):
            @pl.loop(0, QC, step=L)
            def _(i, gir=gir, ch=ch):
                gir[pl.ds(i, L)] = permL[pl.ds(ch * QC + i, L)] >> 1
        gathers = [
            pltpu.async_copy(data_hbm.at[b].at[gir],
                             drows.at[pl.ds(ch * QC, QC), :], gsem)
            for ch, gir in enumerate(girefs)
        ]

        for a in range(4):
            pltpu.sync_copy(p_hbm.at[pl.ds(b * (4 * N) + a * N, N)],
                            pv.at[pl.ds(a * NP, N)])
        pltpu.sync_copy(cs_hbm.at[pl.ds(b * NCELL_PAD, NCELL_PAD)], csv)
        pltpu.sync_copy(mf_hbm.at[pl.ds(b * 128, 128)], mfs)
        pltpu.sync_copy(mi_hbm.at[pl.ds(b * 128, 128)], mis)
        for a in range(NDIM):
            pltpu.sync_copy(
                q_hbm.at[pl.ds(b * (NDIM * M) + a * M + qbase, CHUNK)],
                qv.at[pl.ds(a * CHUNK, CHUNK)])

        mv = mfs[pl.ds(0, L)]
        miv = mis[pl.ds(0, L)]
        lox = mv[0]
        loy = mv[1]
        loz = mv[2]
        gx = miv[0]
        gy = miv[1]
        gz = miv[2]
        sx = miv[3]
        sy = miv[4]

        one = jnp.full((L,), 1, jnp.int32)
        zero = jnp.zeros((L,), jnp.int32)

        def prep(i, _):
            x = qv[pl.ds(0 * CHUNK + i * L, L)]
            y = qv[pl.ds(1 * CHUNK + i * L, L)]
            z = qv[pl.ds(2 * CHUNK + i * L, L)]
            bq[pl.ds(0 * CP + i * L, L)] = _bf16_round(x)
            bq[pl.ds(1 * CP + i * L, L)] = _bf16_round(y)
            bq[pl.ds(2 * CP + i * L, L)] = _bf16_round(z)
            bq[pl.ds(3 * CP + i * L, L)] = (x * x + y * y) + z * z
            for a, w in ((0, x - lox), (1, y - loy), (2, z - loz)):
                t = w * _TEN
                ti = lax.convert_element_type(t, jnp.int32)
                tif = lax.convert_element_type(ti, jnp.float32)
                cq[pl.ds(a * CP + i * L, L)] = ti - jnp.where(t < tif, one,
                                                              zero)
                uq[pl.ds(a * CP + i * L, L)] = t
            return 0
        lax.fori_loop(0, CHUNK // L, prep, 0)

        neg1 = jnp.full((L,), -1.0, jnp.float32)
        r2v = jnp.full((L,), _R2, jnp.float32)

        def do_query(q):
            i3m = jnp.minimum(it, 2)
            qfv = jnp.full((L,), q, jnp.int32)
            qb4 = plsc.load_gather(bq, [qfv + CP * jnp.minimum(it, 3)])
            cq3 = plsc.load_gather(cq, [qfv + CP * i3m])
            uq3 = plsc.load_gather(uq, [qfv + CP * i3m])
            bqx = jnp.full((L,), qb4[0], jnp.float32)
            bqy = jnp.full((L,), qb4[1], jnp.float32)
            bqz = jnp.full((L,), qb4[2], jnp.float32)
            qqv = jnp.full((L,), qb4[3], jnp.float32)
            cqx = cq3[0]
            cqy = cq3[1]
            cqz = cq3[2]

            @pl.loop(0, BUFN, step=L)
            def _(i):
                buf[pl.ds(i, L)] = neg1

            tc2 = jnp.full((L,), np.float32(3.38), jnp.float32)
            uxv = jnp.full((L,), uq3[0], jnp.float32)
            uyv = jnp.full((L,), uq3[1], jnp.float32)
            uzv = jnp.full((L,), uq3[2], jnp.float32)
            czv = jnp.full((L,), cqz, jnp.int32)
            zoks = jnp.logical_and(cqz + 2 >= 0, cqz - 2 <= gz - 1)
            fone = jnp.full((L,), np.float32(1.0), jnp.float32)
            fzero = jnp.zeros((L,), jnp.float32)

            for h in range(2):
                rr = it + h * L
                rx = rr // 5 - 2
                ry = rr % 5 - 2
                cx = jnp.full((L,), cqx, jnp.int32) + rx
                cy = jnp.full((L,), cqy, jnp.int32) + ry
                cxf = lax.convert_element_type(cx, jnp.float32)
                cyf = lax.convert_element_type(cy, jnp.float32)
                dxm = jnp.maximum(
                    jnp.maximum(cxf - uxv, uxv - cxf - fone), fzero)
                dym = jnp.maximum(
                    jnp.maximum(cyf - uyv, uyv - cyf - fone), fzero)
                rxy2 = dxm * dxm + dym * dym
                valid = jnp.logical_and(
                    jnp.logical_and(rr < 25, rxy2 <= tc2),
                    jnp.logical_and(
                        jnp.logical_and(cx >= 0, cx < jnp.full((L,), gx,
                                                            jnp.int32)),
                        jnp.logical_and(cy >= 0, cy < jnp.full((L,), gy,
                                                            jnp.int32))))
                valid = jnp.logical_and(valid, zoks)
                thr = tc2 - rxy2

                def zfail(zc):
                    czf = lax.convert_element_type(zc, jnp.float32)
                    dz = jnp.maximum(
                        jnp.maximum(czf - uzv, uzv - czf - fone), fzero)
                    return lax.convert_element_type(dz * dz > thr,
                                                    jnp.int32)

                zlo_r = jnp.maximum(czv - 2 + zfail(czv - 2)
                                    + zfail(czv - 1), zero)
                zhi_r = jnp.minimum(czv + 2 - zfail(czv + 2)
                                    - zfail(czv + 1),
                                    jnp.full((L,), gz - 1, jnp.int32))
                cxc = jnp.minimum(jnp.maximum(cx, zero),
                                  jnp.full((L,), gx - 1, jnp.int32))
                cyc = jnp.minimum(jnp.maximum(cy, zero),
                                  jnp.full((L,), gy - 1, jnp.int32))
                bc = cxc * jnp.full((L,), sx, jnp.int32) \
                    + cyc * jnp.full((L,), sy, jnp.int32)
                valid = jnp.logical_and(valid, zhi_r >= zlo_r)
                sidx = bc + jnp.maximum(zlo_r, zero)
                eidx = bc + jnp.minimum(zhi_r,
                                        jnp.full((L,), GMAX - 1,
                                                 jnp.int32)) + 1
                s0v = plsc.load_gather(csv, [sidx])
                e0v = plsc.load_gather(csv, [eidx])
                e0v = jnp.where(valid, e0v, s0v)
                ii = jnp.full((L,), 2 * h * L, jnp.int32) + 2 * it
                plsc.store_scatter(rows_se, [ii], s0v)
                plsc.store_scatter(rows_se, [ii + 1], e0v)

            def row(r, cnt):
                v = rows_se[pl.ds(2 * r, L)]
                s0 = v[0]
                e = v[1]

                def cond(carry):
                    return carry[0] < e

                def body(carry):
                    j, c = carry
                    rem = e - j
                    lmask = it < jnp.full((L,), rem, jnp.int32)
                    xv = pv[pl.ds(0 * NP + j, L)]
                    yv = pv[pl.ds(1 * NP + j, L)]
                    zv = pv[pl.ds(2 * NP + j, L)]
                    llv = pv[pl.ds(3 * NP + j, L)]
                    mm = (bqx * xv + bqy * yv) + bqz * zv
                    d2 = (qqv + llv) - (np.float32(2.0) * mm)
                    hit = jnp.logical_and(d2 <= r2v, lmask)
                    pc = plsc.all_reduce_population_count(hit)[0]

                    @pl.when(c < MAX_COLLISIONS)
                    def _():
                        posv = lax.convert_element_type(
                            jnp.full((L,), j, jnp.int32) + it, jnp.float32)
                        plsc.store_compressed(buf.at[pl.ds(c, L)], posv,
                                              mask=hit)
                    return j + L, c + pc
                _, cnt = lax.while_loop(cond, body, (s0, cnt))
                return cnt
            lax.fori_loop(0, 25, row, np.int32(0))

        def group(qc, _):
            def one_q(qs, _):
                q = qc * QG + qs
                do_query(q)

                @pl.loop(0, MAX_COLLISIONS, step=L)
                def _(i):
                    stg[pl.ds(qs * MAX_COLLISIONS + i, L)] = buf[pl.ds(i, L)]
                return 0
            lax.fori_loop(0, QG, one_q, 0)
            pltpu.sync_copy(
                stg,
                nb_hbm.at[pl.ds(
                    b * (M * MAX_COLLISIONS)
                    + (qbase + qc * QG) * MAX_COLLISIONS,
                    QG * MAX_COLLISIONS)])
            return 0
        lax.fori_loop(0, CHUNK // QG, group, 0)

        for gcp in gathers:
            gcp.wait()

        for ch2 in range(4):
            r0 = ch2 * (DC // 4)

            @pl.loop(r0, r0 + DC // 4)
            def _(i, r0=r0):
                h = (_rd(permL, i) & 1) * C
                for kq in range(C // L):
                    v = plsc.load_gather(
                        drows, [jnp.full((L,), i, jnp.int32),
                                jnp.full((L,), h + kq * L, jnp.int32) + it])
                    bounce[pl.ds((i - r0) * C + kq * L, L)] = v

            pltpu.sync_copy(
                bounce,
                datar_hbm.at[pl.ds(b * (N * C) + (dbase + r0) * C,
                                   (DC // 4) * C)])

    return k(qlocs_T, data2, permi, p_arr, cs, mf, mi)


def kernel(locs, data, qlocs):
    B, N, _ = locs.shape
    M = qlocs.shape[1]
    locs_T = jnp.transpose(locs, (0, 2, 1)).reshape(B * NDIM * N)
    qlocs_T = jnp.transpose(qlocs, (0, 2, 1)).reshape(B * NDIM * M)
    C = data.shape[2]
    data2 = data.reshape(B, (N * C) // 128, 128)
    locsr_T, idxs, permi, p_arr, cs, mf, mi = _phase1(locs_T, B)
    neighbors, data_r = _phase2(qlocs_T, data2, permi, p_arr, cs, mf, mi, C)
    locs_r = jnp.transpose(locsr_T.reshape(B, NDIM, N), (0, 2, 1))
    return (locs_r, data_r.reshape(B, N, C), idxs.reshape(B, N),
            neighbors.reshape(B, M, MAX_COLLISIONS))


# speedup vs baseline: 1.2096x; 1.1484x over previous
import dataclasses
import functools

import jax
import jax.numpy as jnp
import numpy as np
from jax import lax
from jax.experimental import pallas as pl
from jax.experimental.pallas import tpu as pltpu
from jax.experimental.pallas import tpu_sc as plsc

NDIM = 3
MAX_COLLISIONS = 128
L = 16
NSUB = 16
GMAX = 12
NCELL = GMAX * GMAX * GMAX
NCELL_PAD = 1792
BUFN = 160

_R2 = np.float32(0.01)
_TEN = np.float32(10.0)
_EPS = np.float32(-1e-6)


def _bf16_round(x):
    u = plsc.bitcast(x, jnp.uint32)
    r = u + np.uint32(0x7FFF) + ((u >> np.uint32(16)) & np.uint32(1))
    r = r & np.uint32(0xFFFF0000)
    return plsc.bitcast(r, jnp.float32)


def _iota16():
    return lax.iota(jnp.int32, L)


def _compiler_params():
    cp = pltpu.CompilerParams()
    if "needs_layout_passes" in pltpu.CompilerParams.__dataclass_fields__:
        cp = dataclasses.replace(cp, needs_layout_passes=False)
    return cp


def _st1(ref, idx, val):
    lane0 = _iota16() == 0
    plsc.store_scatter(ref, [jnp.full((L,), idx, jnp.int32)],
                       jnp.full((L,), val), mask=lane0)


def _rd(ref, idx):
    return ref[pl.ds(idx, L)][0]


def _phase1(locs_T, B):
    N = locs_T.shape[0] // (B * NDIM)
    CHUNK = N // NSUB
    NV = CHUNK // L
    QC = CHUNK // 4
    mesh = plsc.VectorSubcoreMesh(core_axis_name="c", subcore_axis_name="s")

    out_type = (
        jax.ShapeDtypeStruct((B * NDIM * N,), jnp.float32),
        jax.ShapeDtypeStruct((B * N,), jnp.float32),
        jax.ShapeDtypeStruct((B * N,), jnp.int32),
        jax.ShapeDtypeStruct((B * 4 * N,), jnp.float32),
        jax.ShapeDtypeStruct((B * NCELL_PAD,), jnp.int32),
        jax.ShapeDtypeStruct((B * 128,), jnp.float32),
        jax.ShapeDtypeStruct((B * 128,), jnp.int32),
    )
    scratch = [
        pltpu.VMEM((NDIM * CHUNK,), jnp.float32),
        pltpu.VMEM((4 * CHUNK,), jnp.float32),
        pltpu.VMEM((CHUNK,), jnp.int32),
        pltpu.VMEM((QC,), jnp.int32),
        pltpu.VMEM((QC,), jnp.int32),
        pltpu.VMEM((QC,), jnp.int32),
        pltpu.VMEM((QC,), jnp.int32),
        pltpu.VMEM((CHUNK,), jnp.float32),
        pltpu.VMEM((CHUNK,), jnp.int32),
        pltpu.VMEM((NCELL_PAD,), jnp.int32),
        pltpu.VMEM((L,), jnp.float32),
        pltpu.VMEM((NSUB * NCELL_PAD,), jnp.int32),
        pltpu.VMEM((NSUB * NCELL_PAD,), jnp.int32),
        pltpu.VMEM((NCELL_PAD,), jnp.int32),
        pltpu.VMEM((128,), jnp.float32),
        pltpu.VMEM((128,), jnp.int32),
        pltpu.VMEM((NSUB * L,), jnp.float32),
        pltpu.VMEM_SHARED((NSUB * L,), jnp.float32),
        pltpu.VMEM_SHARED((NSUB * NCELL_PAD,), jnp.int32),
        pltpu.VMEM_SHARED((NSUB * NCELL_PAD,), jnp.int32),
        pltpu.VMEM_SHARED((N,), jnp.float32),
        pltpu.VMEM_SHARED((N,), jnp.float32),
        pltpu.VMEM_SHARED((N,), jnp.float32),
        pltpu.VMEM_SHARED((N,), jnp.float32),
        pltpu.VMEM_SHARED((N,), jnp.float32),
        pltpu.VMEM_SHARED((N,), jnp.float32),
        pltpu.VMEM_SHARED((N,), jnp.float32),
        pltpu.VMEM_SHARED((N,), jnp.int32),
        pltpu.SemaphoreType.DMA,
    ]

    @functools.partial(pl.kernel, out_type=out_type, mesh=mesh,
                       scratch_types=scratch,
                       compiler_params=_compiler_params())
    def k(locs_hbm, locsr_hbm, idxs_hbm, permi_hbm, p_hbm,
          cs_hbm, mf_hbm, mi_hbm, xyz, bll, cid, pos0, pos1, pos2, pos3,
          orig, orig_i, hist, mmv, hall, oall, csl, mfs, mis, mm_all,
          sh_mm, sh_hist, sh_off, sh_xr, sh_yr, sh_zr, sh_bx, sh_by,
          sh_bz, sh_ll, sh_oi, sem):
        b = lax.axis_index("c")
        s = lax.axis_index("s")
        base = s * CHUNK
        it = _iota16()

        for a in range(NDIM):
            pltpu.sync_copy(
                locs_hbm.at[pl.ds(b * (NDIM * N) + a * N + base, CHUNK)],
                xyz.at[pl.ds(a * CHUNK, CHUNK)])

        inf = jnp.full((L,), jnp.inf, jnp.float32)
        vec = inf
        for a in range(NDIM):
            def red(i, carry):
                mn, nmx = carry
                v = xyz[pl.ds(a * CHUNK + i * L, L)]
                return jnp.minimum(mn, v), jnp.minimum(nmx, -v)
            mn, nmx = lax.fori_loop(0, NV, red, (inf, inf))
            vec = jnp.where(it == a, jnp.full((L,), jnp.min(mn)), vec)
            vec = jnp.where(it == NDIM + a,
                            jnp.full((L,), jnp.min(nmx)), vec)
        mmv[...] = vec
        pltpu.sync_copy(mmv, sh_mm.at[pl.ds(s * L, L)])
        plsc.subcore_barrier()

        pltpu.sync_copy(sh_mm, mm_all)
        acc = inf
        for t in range(NSUB):
            acc = jnp.minimum(acc, mm_all[pl.ds(t * L, L)])
        mmv[...] = acc
        i3 = jnp.minimum(it, 2)
        mn3 = plsc.load_gather(mmv, [i3])
        nmx3 = plsc.load_gather(mmv, [i3 + 3])
        lower = mn3 + _EPS
        delta = (np.float32(0.0) - nmx3) - lower
        t = delta * _TEN
        ti = lax.convert_element_type(t, jnp.int32)
        tif = lax.convert_element_type(ti, jnp.float32)
        one = jnp.full((L,), 1, jnp.int32)
        zero = jnp.zeros((L,), jnp.int32)
        g = jnp.minimum(ti + jnp.where(t > tif, one, zero) + 1, 96)
        lox = lower[0]
        loy = lower[1]
        loz = lower[2]
        gx = g[0]
        gy = g[1]
        gz = g[2]
        sy = gz
        sx = gy * gz

        gxv = jnp.full((L,), gx - 1, jnp.int32)
        gyv = jnp.full((L,), gy - 1, jnp.int32)
        gzv = jnp.full((L,), gz - 1, jnp.int32)
        sxv = jnp.full((L,), sx, jnp.int32)
        syv = jnp.full((L,), sy, jnp.int32)

        def cells(i, _):
            sl = pl.ds(i * L, L)
            x = xyz[pl.ds(0 * CHUNK + i * L, L)]
            y = xyz[pl.ds(1 * CHUNK + i * L, L)]
            z = xyz[pl.ds(2 * CHUNK + i * L, L)]
            cx = lax.convert_element_type((x - lox) * _TEN, jnp.int32)
            cy = lax.convert_element_type((y - loy) * _TEN, jnp.int32)
            cz = lax.convert_element_type((z - loz) * _TEN, jnp.int32)
            cx = jnp.minimum(jnp.maximum(cx, zero), gxv)
            cy = jnp.minimum(jnp.maximum(cy, zero), gyv)
            cz = jnp.minimum(jnp.maximum(cz, zero), gzv)
            cid[sl] = cx * sxv + cy * syv + cz
            bll[pl.ds(0 * CHUNK + i * L, L)] = _bf16_round(x)
            bll[pl.ds(1 * CHUNK + i * L, L)] = _bf16_round(y)
            bll[pl.ds(2 * CHUNK + i * L, L)] = _bf16_round(z)
            bll[pl.ds(3 * CHUNK + i * L, L)] = (x * x + y * y) + z * z
            oi = jnp.full((L,), base + i * L, jnp.int32) + it
            orig_i[sl] = oi
            orig[sl] = lax.convert_element_type(oi, jnp.float32)
            return 0
        lax.fori_loop(0, NV, cells, 0)

        @pl.loop(0, NCELL_PAD, step=L)
        def _(i):
            hist[pl.ds(i, L)] = zero

        @pl.loop(0, CHUNK, step=L)
        def _(i):
            cv = cid[pl.ds(i, L)]
            for kk in range(L):
                c = cv[kk]
                _st1(hist, c, _rd(hist, c) + 1)

        pltpu.sync_copy(hist, sh_hist.at[pl.ds(s * NCELL_PAD, NCELL_PAD)])
        plsc.subcore_barrier()

        @pl.when(s == 0)
        def _():
            pltpu.sync_copy(sh_hist, hall)

            @pl.loop(NCELL, NCELL_PAD, step=L)
            def _(i):
                csl[pl.ds(i, L)] = jnp.full((L,), N, jnp.int32)

            def scan(kk, bb):
                kv = it * NCELL_PAD + kk
                cnt = plsc.load_gather(hall, [kv])
                incl = plsc.cumsum(cnt)
                excl = incl - cnt
                plsc.store_scatter(oall, [kv],
                                   jnp.full((L,), bb, jnp.int32) + excl)
                _st1(csl, kk, bb)
                return bb + jnp.sum(cnt)
            lax.fori_loop(0, NCELL, scan, np.int32(0))

            pltpu.sync_copy(oall, sh_off)
            pltpu.sync_copy(csl, cs_hbm.at[pl.ds(b * NCELL_PAD, NCELL_PAD)])
            mfs[pl.ds(0, L)] = lower
            mis[pl.ds(0, L)] = jnp.where(
                it == 3, jnp.full((L,), sx, jnp.int32),
                jnp.where(it == 4, jnp.full((L,), sy, jnp.int32), g))
            pltpu.sync_copy(mfs, mf_hbm.at[pl.ds(b * 128, 128)])
            pltpu.sync_copy(mis, mi_hbm.at[pl.ds(b * 128, 128)])
        plsc.subcore_barrier()

        pltpu.sync_copy(sh_off.at[pl.ds(s * NCELL_PAD, NCELL_PAD)], hist)

        for ch, pref in enumerate((pos0, pos1, pos2, pos3)):
            @pl.loop(ch * QC, (ch + 1) * QC, step=L)
            def _(i, pref=pref, ch=ch):
                cv = cid[pl.ds(i, L)]
                for kk in range(L):
                    c = cv[kk]
                    p = _rd(hist, c)
                    _st1(hist, c, p + 1)
                    _st1(pref, i + kk - ch * QC, p)

        copies = []
        for ch, pref in enumerate((pos0, pos1, pos2, pos3)):
            qs = pl.ds(ch * QC, QC)
            copies.append(pltpu.async_copy(
                orig_i.at[qs], sh_oi.at[pref], sem))
            for a, shr in enumerate((sh_xr, sh_yr, sh_zr)):
                copies.append(pltpu.async_copy(
                    xyz.at[pl.ds(a * CHUNK + ch * QC, QC)],
                    shr.at[pref], sem))
            for a, shr in enumerate((sh_bx, sh_by, sh_bz, sh_ll)):
                copies.append(pltpu.async_copy(
                    bll.at[pl.ds(a * CHUNK + ch * QC, QC)],
                    shr.at[pref], sem))
        for cp in copies:
            cp.wait()
        plsc.subcore_barrier()

        csl2 = pl.ds(base, CHUNK)
        for a, shr in enumerate((sh_xr, sh_yr, sh_zr)):
            pltpu.sync_copy(
                shr.at[csl2],
                locsr_hbm.at[pl.ds(b * (NDIM * N) + a * N + base, CHUNK)])
        for a, shr in enumerate((sh_bx, sh_by, sh_bz, sh_ll)):
            pltpu.sync_copy(
                shr.at[csl2],
                p_hbm.at[pl.ds(b * (4 * N) + a * N + base, CHUNK)])
        pltpu.sync_copy(sh_oi.at[csl2],
                        permi_hbm.at[pl.ds(b * N + base, CHUNK)])
        pltpu.sync_copy(sh_oi.at[csl2], orig_i)

        @pl.loop(0, NV, step=1)
        def _(i):
            orig[pl.ds(i * L, L)] = lax.convert_element_type(
                orig_i[pl.ds(i * L, L)], jnp.float32)
        pltpu.sync_copy(orig, idxs_hbm.at[pl.ds(b * N + base, CHUNK)])

    return k(locs_T)


def _phase2(qlocs_T, data2, permi, p_arr, cs, mf, mi, C):
    B = data2.shape[0]
    M = qlocs_T.shape[0] // (B * NDIM)
    N = p_arr.shape[0] // (B * 4)
    NP = N + L
    CHUNK = M // NSUB
    CP = CHUNK + L
    DC = N // NSUB
    QC = DC // 4
    QG = 32
    mesh = plsc.VectorSubcoreMesh(core_axis_name="c", subcore_axis_name="s")

    out_type = (
        jax.ShapeDtypeStruct((B * M * MAX_COLLISIONS,), jnp.float32),
        jax.ShapeDtypeStruct((B * N * C,), jnp.float32),
    )
    scratch = [
        pltpu.VMEM((4 * NP,), jnp.float32),
        pltpu.VMEM((NCELL_PAD,), jnp.int32),
        pltpu.VMEM((NDIM * CHUNK,), jnp.float32),
        pltpu.VMEM((5 * CP,), jnp.float32),
        pltpu.VMEM((NDIM * CP,), jnp.int32),
        pltpu.VMEM((NDIM * CP,), jnp.float32),
        pltpu.VMEM((4 * L,), jnp.int32),
        pltpu.VMEM((4 * L,), jnp.int32),
        pltpu.VMEM((BUFN,), jnp.float32),
        pltpu.VMEM((QG * MAX_COLLISIONS,), jnp.float32),
        pltpu.VMEM((128,), jnp.float32),
        pltpu.VMEM((128,), jnp.int32),
        pltpu.VMEM((QC,), jnp.int32),
        pltpu.VMEM((QC,), jnp.int32),
        pltpu.VMEM((QC,), jnp.int32),
        pltpu.VMEM((QC,), jnp.int32),
        pltpu.VMEM((DC + L,), jnp.int32),
        pltpu.VMEM((DC, 128), jnp.float32),
        pltpu.VMEM((128 * 64,), jnp.float32),
        pltpu.SemaphoreType.DMA,
        pltpu.SemaphoreType.DMA,
    ]

    @functools.partial(pl.kernel, out_type=out_type, mesh=mesh,
                       scratch_types=scratch,
                       compiler_params=_compiler_params())
    def k(q_hbm, data_hbm, permi_hbm, p_hbm, cs_hbm, mf_hbm, mi_hbm,
          nb_hbm, datar_hbm, pv, csv, qv, bq, cq, uq, rows_s, rows_e,
          buf, stg, mfs, mis, gi0, gi1, gi2, gi3, permL, drows, bounce,
          sem, gsem):
        b = lax.axis_index("c")
        s = lax.axis_index("s")
        qbase = s * CHUNK
        dbase = s * DC
        it = _iota16()

        pltpu.sync_copy(permi_hbm.at[pl.ds(b * N + dbase, DC)],
                        permL.at[pl.ds(0, DC)])
        girefs = (gi0, gi1, gi2, gi3)
        for ch, gir in enumerate(girefs):
            @pl.loop(0, QC, step=L)
            def _(i, gir=gir, ch=ch):
                gir[pl.ds(i, L)] = permL[pl.ds(ch * QC + i, L)] >> 1
        gathers = [
            pltpu.async_copy(data_hbm.at[b].at[gir],
                             drows.at[pl.ds(ch * QC, QC), :], gsem)
            for ch, gir in enumerate(girefs)
        ]

        for a in range(4):
            pltpu.sync_copy(p_hbm.at[pl.ds(b * (4 * N) + a * N, N)],
                            pv.at[pl.ds(a * NP, N)])
        pltpu.sync_copy(cs_hbm.at[pl.ds(b * NCELL_PAD, NCELL_PAD)], csv)
        pltpu.sync_copy(mf_hbm.at[pl.ds(b * 128, 128)], mfs)
        pltpu.sync_copy(mi_hbm.at[pl.ds(b * 128, 128)], mis)
        for a in range(NDIM):
            pltpu.sync_copy(
                q_hbm.at[pl.ds(b * (NDIM * M) + a * M + qbase, CHUNK)],
                qv.at[pl.ds(a * CHUNK, CHUNK)])

        mv = mfs[pl.ds(0, L)]
        miv = mis[pl.ds(0, L)]
        lox = mv[0]
        loy = mv[1]
        loz = mv[2]
        gx = miv[0]
        gy = miv[1]
        gz = miv[2]
        sx = miv[3]
        sy = miv[4]

        one = jnp.full((L,), 1, jnp.int32)
        zero = jnp.zeros((L,), jnp.int32)

        def prep(i, _):
            x = qv[pl.ds(0 * CHUNK + i * L, L)]
            y = qv[pl.ds(1 * CHUNK + i * L, L)]
            z = qv[pl.ds(2 * CHUNK + i * L, L)]
            bx = _bf16_round(x)
            by = _bf16_round(y)
            bz = _bf16_round(z)
            bq[pl.ds(0 * CP + i * L, L)] = bx
            bq[pl.ds(1 * CP + i * L, L)] = by
            bq[pl.ds(2 * CP + i * L, L)] = bz
            bq[pl.ds(3 * CP + i * L, L)] = (x * x + y * y) + z * z
            c8 = np.float32(0.00396)
            fone_ = jnp.full((L,), np.float32(1.0), jnp.float32)
            r24 = np.float32(0.24)
            s2 = (jnp.minimum(fone_, x + r24) * (jnp.abs(x - bx) + bx * c8)
                  + jnp.minimum(fone_, y + r24) * (jnp.abs(y - by) + by * c8)
                  + jnp.minimum(fone_, z + r24) * (jnp.abs(z - bz) + bz * c8))
            bq[pl.ds(4 * CP + i * L, L)] = (jnp.full((L,), np.float32(1.01))
                                            + np.float32(200.8) * s2)
            for a, w in ((0, x - lox), (1, y - loy), (2, z - loz)):
                t = w * _TEN
                ti = lax.convert_element_type(t, jnp.int32)
                tif = lax.convert_element_type(ti, jnp.float32)
                cq[pl.ds(a * CP + i * L, L)] = ti - jnp.where(t < tif, one,
                                                              zero)
                uq[pl.ds(a * CP + i * L, L)] = t
            return 0
        lax.fori_loop(0, CHUNK // L, prep, 0)

        neg1 = jnp.full((L,), -1.0, jnp.float32)
        r2v = jnp.full((L,), _R2, jnp.float32)

        def do_query(q):
            i3m = jnp.minimum(it, 2)
            qfv = jnp.full((L,), q, jnp.int32)
            qb4 = plsc.load_gather(bq, [qfv + CP * jnp.minimum(it, 4)])
            cq3 = plsc.load_gather(cq, [qfv + CP * i3m])
            uq3 = plsc.load_gather(uq, [qfv + CP * i3m])
            bqx = jnp.full((L,), qb4[0], jnp.float32)
            bqy = jnp.full((L,), qb4[1], jnp.float32)
            bqz = jnp.full((L,), qb4[2], jnp.float32)
            qqv = jnp.full((L,), qb4[3], jnp.float32)
            cqx = cq3[0]
            cqy = cq3[1]
            cqz = cq3[2]

            @pl.loop(0, BUFN, step=L)
            def _(i):
                buf[pl.ds(i, L)] = neg1

            tc2 = jnp.full((L,), qb4[4], jnp.float32)
            uxv = jnp.full((L,), uq3[0], jnp.float32)
            uyv = jnp.full((L,), uq3[1], jnp.float32)
            uzv = jnp.full((L,), uq3[2], jnp.float32)
            czv = jnp.full((L,), cqz, jnp.int32)
            zoks = jnp.logical_and(cqz + 2 >= 0, cqz - 2 <= gz - 1)
            fone = jnp.full((L,), np.float32(1.0), jnp.float32)
            fzero = jnp.zeros((L,), jnp.float32)

            for h in range(2):
                rr = it + h * L
                rx = rr // 5 - 2
                ry = rr % 5 - 2
                cx = jnp.full((L,), cqx, jnp.int32) + rx
                cy = jnp.full((L,), cqy, jnp.int32) + ry
                cxf = lax.convert_element_type(cx, jnp.float32)
                cyf = lax.convert_element_type(cy, jnp.float32)
                dxm = jnp.maximum(
                    jnp.maximum(cxf - uxv, uxv - cxf - fone), fzero)
                dym = jnp.maximum(
                    jnp.maximum(cyf - uyv, uyv - cyf - fone), fzero)
                rxy2 = dxm * dxm + dym * dym
                valid = jnp.logical_and(
                    jnp.logical_and(rr < 25, rxy2 <= tc2),
                    jnp.logical_and(
                        jnp.logical_and(cx >= 0, cx < jnp.full((L,), gx,
                                                            jnp.int32)),
                        jnp.logical_and(cy >= 0, cy < jnp.full((L,), gy,
                                                            jnp.int32))))
                valid = jnp.logical_and(valid, zoks)
                thr = tc2 - rxy2

                def zfail(zc):
                    czf = lax.convert_element_type(zc, jnp.float32)
                    dz = jnp.maximum(
                        jnp.maximum(czf - uzv, uzv - czf - fone), fzero)
                    return lax.convert_element_type(dz * dz > thr,
                                                    jnp.int32)

                zlo_r = jnp.maximum(czv - 2 + zfail(czv - 2)
                                    + zfail(czv - 1), zero)
                zhi_r = jnp.minimum(czv + 2 - zfail(czv + 2)
                                    - zfail(czv + 1),
                                    jnp.full((L,), gz - 1, jnp.int32))
                cxc = jnp.minimum(jnp.maximum(cx, zero),
                                  jnp.full((L,), gx - 1, jnp.int32))
                cyc = jnp.minimum(jnp.maximum(cy, zero),
                                  jnp.full((L,), gy - 1, jnp.int32))
                bc = cxc * jnp.full((L,), sx, jnp.int32) \
                    + cyc * jnp.full((L,), sy, jnp.int32)
                valid = jnp.logical_and(valid, zhi_r >= zlo_r)
                sidx = bc + jnp.maximum(zlo_r, zero)
                eidx = bc + jnp.minimum(zhi_r,
                                        jnp.full((L,), GMAX - 1,
                                                 jnp.int32)) + 1
                s0v = plsc.load_gather(csv, [sidx])
                e0v = plsc.load_gather(csv, [eidx])
                e0v = jnp.where(valid, e0v, s0v)
                rows_s[pl.ds(h * L, L)] = s0v
                rows_e[pl.ds(h * L, L)] = e0v

            def row(r, cnt):
                s0 = _rd(rows_s, r)
                e = _rd(rows_e, r)

                def cond(carry):
                    return carry[0] < e

                def body(carry):
                    j, c = carry
                    rem = e - j
                    lmask = it < jnp.full((L,), rem, jnp.int32)
                    xv = pv[pl.ds(0 * NP + j, L)]
                    yv = pv[pl.ds(1 * NP + j, L)]
                    zv = pv[pl.ds(2 * NP + j, L)]
                    llv = pv[pl.ds(3 * NP + j, L)]
                    mm = (bqx * xv + bqy * yv) + bqz * zv
                    d2 = (qqv + llv) - (np.float32(2.0) * mm)
                    hit = jnp.logical_and(d2 <= r2v, lmask)
                    pc = plsc.all_reduce_population_count(hit)[0]

                    @pl.when(c < MAX_COLLISIONS)
                    def _():
                        posv = lax.convert_element_type(
                            jnp.full((L,), j, jnp.int32) + it, jnp.float32)
                        plsc.store_compressed(buf.at[pl.ds(c, L)], posv,
                                              mask=hit)
                    return j + L, c + pc
                _, cnt = lax.while_loop(cond, body, (s0, cnt))
                return cnt
            lax.fori_loop(0, 25, row, np.int32(0))

        def group(qc, _):
            def one_q(qs, _):
                q = qc * QG + qs
                do_query(q)

                @pl.loop(0, MAX_COLLISIONS, step=L)
                def _(i):
                    stg[pl.ds(qs * MAX_COLLISIONS + i, L)] = buf[pl.ds(i, L)]
                return 0
            lax.fori_loop(0, QG, one_q, 0)
            pltpu.sync_copy(
                stg,
                nb_hbm.at[pl.ds(
                    b * (M * MAX_COLLISIONS)
                    + (qbase + qc * QG) * MAX_COLLISIONS,
                    QG * MAX_COLLISIONS)])
            return 0
        lax.fori_loop(0, CHUNK // QG, group, 0)

        for gcp in gathers:
            gcp.wait()

        for ch2 in range(4):
            r0 = ch2 * (DC // 4)

            @pl.loop(r0, r0 + DC // 4)
            def _(i, r0=r0):
                h = (_rd(permL, i) & 1) * C
                for kq in range(C // L):
                    v = plsc.load_gather(
                        drows, [jnp.full((L,), i, jnp.int32),
                                jnp.full((L,), h + kq * L, jnp.int32) + it])
                    bounce[pl.ds((i - r0) * C + kq * L, L)] = v

            pltpu.sync_copy(
                bounce,
                datar_hbm.at[pl.ds(b * (N * C) + (dbase + r0) * C,
                                   (DC // 4) * C)])

    return k(qlocs_T, data2, permi, p_arr, cs, mf, mi)


def kernel(locs, data, qlocs):
    B, N, _ = locs.shape
    M = qlocs.shape[1]
    locs_T = jnp.transpose(locs, (0, 2, 1)).reshape(B * NDIM * N)
    qlocs_T = jnp.transpose(qlocs, (0, 2, 1)).reshape(B * NDIM * M)
    C = data.shape[2]
    data2 = data.reshape(B, (N * C) // 128, 128)
    locsr_T, idxs, permi, p_arr, cs, mf, mi = _phase1(locs_T, B)
    neighbors, data_r = _phase2(qlocs_T, data2, permi, p_arr, cs, mf, mi, C)
    locs_r = jnp.transpose(locsr_T.reshape(B, NDIM, N), (0, 2, 1))
    return (locs_r, data_r.reshape(B, N, C), idxs.reshape(B, N),
            neighbors.reshape(B, M, MAX_COLLISIONS))


# speedup vs baseline: 1.4600x; 1.2070x over previous
import dataclasses
import functools

import jax
import jax.numpy as jnp
import numpy as np
from jax import lax
from jax.experimental import pallas as pl
from jax.experimental.pallas import tpu as pltpu
from jax.experimental.pallas import tpu_sc as plsc

NDIM = 3
MAX_COLLISIONS = 128
L = 16
NSUB = 16
GMAX = 12
NCELL = GMAX * GMAX * GMAX
NCELL_PAD = 1792
BUFN = 160

_R2 = np.float32(0.01)
_TEN = np.float32(10.0)
_EPS = np.float32(-1e-6)


def _bf16_round(x):
    u = plsc.bitcast(x, jnp.uint32)
    r = u + np.uint32(0x7FFF) + ((u >> np.uint32(16)) & np.uint32(1))
    r = r & np.uint32(0xFFFF0000)
    return plsc.bitcast(r, jnp.float32)


def _iota16():
    return lax.iota(jnp.int32, L)


def _compiler_params():
    cp = pltpu.CompilerParams()
    if "needs_layout_passes" in pltpu.CompilerParams.__dataclass_fields__:
        cp = dataclasses.replace(cp, needs_layout_passes=False)
    return cp


def _st1(ref, idx, val):
    lane0 = _iota16() == 0
    plsc.store_scatter(ref, [jnp.full((L,), idx, jnp.int32)],
                       jnp.full((L,), val), mask=lane0)


def _rd(ref, idx):
    return ref[pl.ds(idx, L)][0]


def _phase1(locs_T, B):
    N = locs_T.shape[0] // (B * NDIM)
    CHUNK = N // NSUB
    NV = CHUNK // L
    QC = CHUNK // 4
    mesh = plsc.VectorSubcoreMesh(core_axis_name="c", subcore_axis_name="s")

    out_type = (
        jax.ShapeDtypeStruct((B * NDIM * N,), jnp.float32),
        jax.ShapeDtypeStruct((B * N,), jnp.float32),
        jax.ShapeDtypeStruct((B * N,), jnp.int32),
        jax.ShapeDtypeStruct((B * 4 * N,), jnp.float32),
        jax.ShapeDtypeStruct((B * NCELL_PAD,), jnp.int32),
        jax.ShapeDtypeStruct((B * 128,), jnp.float32),
        jax.ShapeDtypeStruct((B * 128,), jnp.int32),
    )
    scratch = [
        pltpu.VMEM((NDIM * CHUNK,), jnp.float32),
        pltpu.VMEM((4 * CHUNK,), jnp.float32),
        pltpu.VMEM((CHUNK,), jnp.int32),
        pltpu.VMEM((QC,), jnp.int32),
        pltpu.VMEM((QC,), jnp.int32),
        pltpu.VMEM((QC,), jnp.int32),
        pltpu.VMEM((QC,), jnp.int32),
        pltpu.VMEM((CHUNK,), jnp.float32),
        pltpu.VMEM((CHUNK,), jnp.int32),
        pltpu.VMEM((NCELL_PAD,), jnp.int32),
        pltpu.VMEM((L,), jnp.float32),
        pltpu.VMEM((NSUB * NCELL_PAD,), jnp.int32),
        pltpu.VMEM((NSUB * NCELL_PAD,), jnp.int32),
        pltpu.VMEM((NCELL_PAD,), jnp.int32),
        pltpu.VMEM((128,), jnp.float32),
        pltpu.VMEM((128,), jnp.int32),
        pltpu.VMEM((NSUB * L,), jnp.float32),
        pltpu.VMEM_SHARED((NSUB * L,), jnp.float32),
        pltpu.VMEM_SHARED((NSUB * NCELL_PAD,), jnp.int32),
        pltpu.VMEM_SHARED((NSUB * NCELL_PAD,), jnp.int32),
        pltpu.VMEM_SHARED((N,), jnp.float32),
        pltpu.VMEM_SHARED((N,), jnp.float32),
        pltpu.VMEM_SHARED((N,), jnp.float32),
        pltpu.VMEM_SHARED((N,), jnp.float32),
        pltpu.VMEM_SHARED((N,), jnp.float32),
        pltpu.VMEM_SHARED((N,), jnp.float32),
        pltpu.VMEM_SHARED((N,), jnp.float32),
        pltpu.VMEM_SHARED((N,), jnp.int32),
        pltpu.SemaphoreType.DMA,
    ]

    @functools.partial(pl.kernel, out_type=out_type, mesh=mesh,
                       scratch_types=scratch,
                       compiler_params=_compiler_params())
    def k(locs_hbm, locsr_hbm, idxs_hbm, permi_hbm, p_hbm,
          cs_hbm, mf_hbm, mi_hbm, xyz, bll, cid, pos0, pos1, pos2, pos3,
          orig, orig_i, hist, mmv, hall, oall, csl, mfs, mis, mm_all,
          sh_mm, sh_hist, sh_off, sh_xr, sh_yr, sh_zr, sh_bx, sh_by,
          sh_bz, sh_ll, sh_oi, sem):
        b = lax.axis_index("c")
        s = lax.axis_index("s")
        base = s * CHUNK
        it = _iota16()

        for a in range(NDIM):
            pltpu.sync_copy(
                locs_hbm.at[pl.ds(b * (NDIM * N) + a * N + base, CHUNK)],
                xyz.at[pl.ds(a * CHUNK, CHUNK)])

        inf = jnp.full((L,), jnp.inf, jnp.float32)
        vec = inf
        for a in range(NDIM):
            def red(i, carry):
                mn, nmx = carry
                v = xyz[pl.ds(a * CHUNK + i * L, L)]
                return jnp.minimum(mn, v), jnp.minimum(nmx, -v)
            mn, nmx = lax.fori_loop(0, NV, red, (inf, inf))
            vec = jnp.where(it == a, jnp.full((L,), jnp.min(mn)), vec)
            vec = jnp.where(it == NDIM + a,
                            jnp.full((L,), jnp.min(nmx)), vec)
        mmv[...] = vec
        pltpu.sync_copy(mmv, sh_mm.at[pl.ds(s * L, L)])
        plsc.subcore_barrier()

        pltpu.sync_copy(sh_mm, mm_all)
        acc = inf
        for t in range(NSUB):
            acc = jnp.minimum(acc, mm_all[pl.ds(t * L, L)])
        mmv[...] = acc
        i3 = jnp.minimum(it, 2)
        mn3 = plsc.load_gather(mmv, [i3])
        nmx3 = plsc.load_gather(mmv, [i3 + 3])
        lower = mn3 + _EPS
        delta = (np.float32(0.0) - nmx3) - lower
        t = delta * _TEN
        ti = lax.convert_element_type(t, jnp.int32)
        tif = lax.convert_element_type(ti, jnp.float32)
        one = jnp.full((L,), 1, jnp.int32)
        zero = jnp.zeros((L,), jnp.int32)
        g = jnp.minimum(ti + jnp.where(t > tif, one, zero) + 1, 96)
        lox = lower[0]
        loy = lower[1]
        loz = lower[2]
        gx = g[0]
        gy = g[1]
        gz = g[2]
        sy = gz
        sx = gy * gz

        gxv = jnp.full((L,), gx - 1, jnp.int32)
        gyv = jnp.full((L,), gy - 1, jnp.int32)
        gzv = jnp.full((L,), gz - 1, jnp.int32)
        sxv = jnp.full((L,), sx, jnp.int32)
        syv = jnp.full((L,), sy, jnp.int32)

        def cells(i, _):
            sl = pl.ds(i * L, L)
            x = xyz[pl.ds(0 * CHUNK + i * L, L)]
            y = xyz[pl.ds(1 * CHUNK + i * L, L)]
            z = xyz[pl.ds(2 * CHUNK + i * L, L)]
            cx = lax.convert_element_type((x - lox) * _TEN, jnp.int32)
            cy = lax.convert_element_type((y - loy) * _TEN, jnp.int32)
            cz = lax.convert_element_type((z - loz) * _TEN, jnp.int32)
            cx = jnp.minimum(jnp.maximum(cx, zero), gxv)
            cy = jnp.minimum(jnp.maximum(cy, zero), gyv)
            cz = jnp.minimum(jnp.maximum(cz, zero), gzv)
            cid[sl] = cx * sxv + cy * syv + cz
            bll[pl.ds(0 * CHUNK + i * L, L)] = _bf16_round(x)
            bll[pl.ds(1 * CHUNK + i * L, L)] = _bf16_round(y)
            bll[pl.ds(2 * CHUNK + i * L, L)] = _bf16_round(z)
            bll[pl.ds(3 * CHUNK + i * L, L)] = (x * x + y * y) + z * z
            oi = jnp.full((L,), base + i * L, jnp.int32) + it
            orig_i[sl] = oi
            orig[sl] = lax.convert_element_type(oi, jnp.float32)
            return 0
        lax.fori_loop(0, NV, cells, 0)

        @pl.loop(0, NCELL_PAD, step=L)
        def _(i):
            hist[pl.ds(i, L)] = zero

        @pl.loop(0, CHUNK, step=L)
        def _(i):
            cv = cid[pl.ds(i, L)]
            for kk in range(L):
                c = cv[kk]
                _st1(hist, c, _rd(hist, c) + 1)

        pltpu.sync_copy(hist, sh_hist.at[pl.ds(s * NCELL_PAD, NCELL_PAD)])
        plsc.subcore_barrier()

        @pl.when(s == 0)
        def _():
            pltpu.sync_copy(sh_hist, hall)

            @pl.loop(NCELL, NCELL_PAD, step=L)
            def _(i):
                csl[pl.ds(i, L)] = jnp.full((L,), N, jnp.int32)

            def scan(kk, bb):
                kv = it * NCELL_PAD + kk
                cnt = plsc.load_gather(hall, [kv])
                incl = plsc.cumsum(cnt)
                excl = incl - cnt
                plsc.store_scatter(oall, [kv],
                                   jnp.full((L,), bb, jnp.int32) + excl)
                _st1(csl, kk, bb)
                return bb + jnp.sum(cnt)
            lax.fori_loop(0, NCELL, scan, np.int32(0))

            pltpu.sync_copy(oall, sh_off)
            pltpu.sync_copy(csl, cs_hbm.at[pl.ds(b * NCELL_PAD, NCELL_PAD)])
            mfs[pl.ds(0, L)] = lower
            mis[pl.ds(0, L)] = jnp.where(
                it == 3, jnp.full((L,), sx, jnp.int32),
                jnp.where(it == 4, jnp.full((L,), sy, jnp.int32), g))
            pltpu.sync_copy(mfs, mf_hbm.at[pl.ds(b * 128, 128)])
            pltpu.sync_copy(mis, mi_hbm.at[pl.ds(b * 128, 128)])
        plsc.subcore_barrier()

        pltpu.sync_copy(sh_off.at[pl.ds(s * NCELL_PAD, NCELL_PAD)], hist)

        for ch, pref in enumerate((pos0, pos1, pos2, pos3)):
            @pl.loop(ch * QC, (ch + 1) * QC, step=L)
            def _(i, pref=pref, ch=ch):
                cv = cid[pl.ds(i, L)]
                for kk in range(L):
                    c = cv[kk]
                    p = _rd(hist, c)
                    _st1(hist, c, p + 1)
                    _st1(pref, i + kk - ch * QC, p)

        copies = []
        for ch, pref in enumerate((pos0, pos1, pos2, pos3)):
            qs = pl.ds(ch * QC, QC)
            copies.append(pltpu.async_copy(
                orig_i.at[qs], sh_oi.at[pref], sem))
            for a, shr in enumerate((sh_xr, sh_yr, sh_zr)):
                copies.append(pltpu.async_copy(
                    xyz.at[pl.ds(a * CHUNK + ch * QC, QC)],
                    shr.at[pref], sem))
            for a, shr in enumerate((sh_bx, sh_by, sh_bz, sh_ll)):
                copies.append(pltpu.async_copy(
                    bll.at[pl.ds(a * CHUNK + ch * QC, QC)],
                    shr.at[pref], sem))
        for cp in copies:
            cp.wait()
        plsc.subcore_barrier()

        csl2 = pl.ds(base, CHUNK)
        for a, shr in enumerate((sh_xr, sh_yr, sh_zr)):
            pltpu.sync_copy(
                shr.at[csl2],
                locsr_hbm.at[pl.ds(b * (NDIM * N) + a * N + base, CHUNK)])
        for a, shr in enumerate((sh_bx, sh_by, sh_bz, sh_ll)):
            pltpu.sync_copy(
                shr.at[csl2],
                p_hbm.at[pl.ds(b * (4 * N) + a * N + base, CHUNK)])
        pltpu.sync_copy(sh_oi.at[csl2],
                        permi_hbm.at[pl.ds(b * N + base, CHUNK)])
        pltpu.sync_copy(sh_oi.at[csl2], orig_i)

        @pl.loop(0, NV, step=1)
        def _(i):
            orig[pl.ds(i * L, L)] = lax.convert_element_type(
                orig_i[pl.ds(i * L, L)], jnp.float32)
        pltpu.sync_copy(orig, idxs_hbm.at[pl.ds(b * N + base, CHUNK)])

    return k(locs_T)


def _phase2(qlocs_T, data2, permi, p_arr, cs, mf, mi, C):
    B = data2.shape[0]
    M = qlocs_T.shape[0] // (B * NDIM)
    N = p_arr.shape[0] // (B * 4)
    NP = N + L
    CHUNK = M // NSUB
    CP = CHUNK + L
    DC = N // NSUB
    QC = DC // 4
    QG = 32
    mesh = plsc.VectorSubcoreMesh(core_axis_name="c", subcore_axis_name="s")

    out_type = (
        jax.ShapeDtypeStruct((B * M * MAX_COLLISIONS,), jnp.float32),
        jax.ShapeDtypeStruct((B * N * C,), jnp.float32),
    )
    scratch = [
        pltpu.VMEM((4 * NP,), jnp.float32),
        pltpu.VMEM((NCELL_PAD,), jnp.int32),
        pltpu.VMEM((NDIM * CHUNK,), jnp.float32),
        pltpu.VMEM((5 * CP,), jnp.float32),
        pltpu.VMEM((NDIM * CP,), jnp.int32),
        pltpu.VMEM((NDIM * CP,), jnp.float32),
        pltpu.VMEM((4 * L,), jnp.int32),
        pltpu.VMEM((4 * L,), jnp.int32),
        pltpu.VMEM((BUFN,), jnp.float32),
        pltpu.VMEM((QG * MAX_COLLISIONS,), jnp.float32),
        pltpu.VMEM((128,), jnp.float32),
        pltpu.VMEM((128,), jnp.int32),
        pltpu.VMEM((QC,), jnp.int32),
        pltpu.VMEM((QC,), jnp.int32),
        pltpu.VMEM((QC,), jnp.int32),
        pltpu.VMEM((QC,), jnp.int32),
        pltpu.VMEM((DC + L,), jnp.int32),
        pltpu.VMEM((DC, 128), jnp.float32),
        pltpu.VMEM((128 * 64,), jnp.float32),
        pltpu.SemaphoreType.DMA,
        pltpu.SemaphoreType.DMA,
    ]

    @functools.partial(pl.kernel, out_type=out_type, mesh=mesh,
                       scratch_types=scratch,
                       compiler_params=_compiler_params())
    def k(q_hbm, data_hbm, permi_hbm, p_hbm, cs_hbm, mf_hbm, mi_hbm,
          nb_hbm, datar_hbm, pv, csv, qv, bq, cq, uq, rows_s, rows_e,
          buf, stg, mfs, mis, gi0, gi1, gi2, gi3, permL, drows, bounce,
          sem, gsem):
        b = lax.axis_index("c")
        s = lax.axis_index("s")
        qbase = s * CHUNK
        dbase = s * DC
        it = _iota16()

        pltpu.sync_copy(permi_hbm.at[pl.ds(b * N + dbase, DC)],
                        permL.at[pl.ds(0, DC)])
        girefs = (gi0, gi1, gi2, gi3)
        for ch, gir in enumerate(girefs):
            @pl.loop(0, QC, step=L)
            def _(i, gir=gir, ch=ch):
                gir[pl.ds(i, L)] = permL[pl.ds(ch * QC + i, L)] >> 1
        gathers = [
            pltpu.async_copy(data_hbm.at[b].at[gir],
                             drows.at[pl.ds(ch * QC, QC), :], gsem)
            for ch, gir in enumerate(girefs)
        ]

        for a in range(4):
            pltpu.sync_copy(p_hbm.at[pl.ds(b * (4 * N) + a * N, N)],
                            pv.at[pl.ds(a * NP, N)])
        pltpu.sync_copy(cs_hbm.at[pl.ds(b * NCELL_PAD, NCELL_PAD)], csv)
        pltpu.sync_copy(mf_hbm.at[pl.ds(b * 128, 128)], mfs)
        pltpu.sync_copy(mi_hbm.at[pl.ds(b * 128, 128)], mis)
        for a in range(NDIM):
            pltpu.sync_copy(
                q_hbm.at[pl.ds(b * (NDIM * M) + a * M + qbase, CHUNK)],
                qv.at[pl.ds(a * CHUNK, CHUNK)])

        mv = mfs[pl.ds(0, L)]
        miv = mis[pl.ds(0, L)]
        lox = mv[0]
        loy = mv[1]
        loz = mv[2]
        gx = miv[0]
        gy = miv[1]
        gz = miv[2]
        sx = miv[3]
        sy = miv[4]

        one = jnp.full((L,), 1, jnp.int32)
        zero = jnp.zeros((L,), jnp.int32)

        def prep(i, _):
            x = qv[pl.ds(0 * CHUNK + i * L, L)]
            y = qv[pl.ds(1 * CHUNK + i * L, L)]
            z = qv[pl.ds(2 * CHUNK + i * L, L)]
            bx = _bf16_round(x)
            by = _bf16_round(y)
            bz = _bf16_round(z)
            bq[pl.ds(0 * CP + i * L, L)] = bx
            bq[pl.ds(1 * CP + i * L, L)] = by
            bq[pl.ds(2 * CP + i * L, L)] = bz
            bq[pl.ds(3 * CP + i * L, L)] = (x * x + y * y) + z * z
            c8 = np.float32(0.00396)
            fone_ = jnp.full((L,), np.float32(1.0), jnp.float32)
            r24 = np.float32(0.24)
            s2 = (jnp.minimum(fone_, x + r24) * (jnp.abs(x - bx) + bx * c8)
                  + jnp.minimum(fone_, y + r24) * (jnp.abs(y - by) + by * c8)
                  + jnp.minimum(fone_, z + r24) * (jnp.abs(z - bz) + bz * c8))
            bq[pl.ds(4 * CP + i * L, L)] = (jnp.full((L,), np.float32(1.01))
                                            + np.float32(200.8) * s2)
            for a, w in ((0, x - lox), (1, y - loy), (2, z - loz)):
                t = w * _TEN
                ti = lax.convert_element_type(t, jnp.int32)
                tif = lax.convert_element_type(ti, jnp.float32)
                cq[pl.ds(a * CP + i * L, L)] = ti - jnp.where(t < tif, one,
                                                              zero)
                uq[pl.ds(a * CP + i * L, L)] = t
            return 0
        lax.fori_loop(0, CHUNK // L, prep, 0)

        neg1 = jnp.full((L,), -1.0, jnp.float32)
        r2v = jnp.full((L,), _R2, jnp.float32)

        def do_query(q):
            i3m = jnp.minimum(it, 2)
            qfv = jnp.full((L,), q, jnp.int32)
            qb4 = plsc.load_gather(bq, [qfv + CP * jnp.minimum(it, 4)])
            cq3 = plsc.load_gather(cq, [qfv + CP * i3m])
            uq3 = plsc.load_gather(uq, [qfv + CP * i3m])
            bqx = jnp.full((L,), qb4[0], jnp.float32)
            bqy = jnp.full((L,), qb4[1], jnp.float32)
            bqz = jnp.full((L,), qb4[2], jnp.float32)
            qqv = jnp.full((L,), qb4[3], jnp.float32)
            cqx = cq3[0]
            cqy = cq3[1]
            cqz = cq3[2]

            @pl.loop(0, BUFN, step=L)
            def _(i):
                buf[pl.ds(i, L)] = neg1

            tc2 = jnp.full((L,), qb4[4], jnp.float32)
            uxv = jnp.full((L,), uq3[0], jnp.float32)
            uyv = jnp.full((L,), uq3[1], jnp.float32)
            uzv = jnp.full((L,), uq3[2], jnp.float32)
            czv = jnp.full((L,), cqz, jnp.int32)
            zoks = jnp.logical_and(cqz + 2 >= 0, cqz - 2 <= gz - 1)
            fone = jnp.full((L,), np.float32(1.0), jnp.float32)
            fzero = jnp.zeros((L,), jnp.float32)
            nr = cqz * 0

            for h in range(2):
                rr = it + h * L
                rx = rr // 5 - 2
                ry = rr % 5 - 2
                cx = jnp.full((L,), cqx, jnp.int32) + rx
                cy = jnp.full((L,), cqy, jnp.int32) + ry
                cxf = lax.convert_element_type(cx, jnp.float32)
                cyf = lax.convert_element_type(cy, jnp.float32)
                dxm = jnp.maximum(
                    jnp.maximum(cxf - uxv, uxv - cxf - fone), fzero)
                dym = jnp.maximum(
                    jnp.maximum(cyf - uyv, uyv - cyf - fone), fzero)
                rxy2 = dxm * dxm + dym * dym
                valid = jnp.logical_and(
                    jnp.logical_and(rr < 25, rxy2 <= tc2),
                    jnp.logical_and(
                        jnp.logical_and(cx >= 0, cx < jnp.full((L,), gx,
                                                            jnp.int32)),
                        jnp.logical_and(cy >= 0, cy < jnp.full((L,), gy,
                                                            jnp.int32))))
                valid = jnp.logical_and(valid, zoks)
                thr = tc2 - rxy2

                def zfail(zc):
                    czf = lax.convert_element_type(zc, jnp.float32)
                    dz = jnp.maximum(
                        jnp.maximum(czf - uzv, uzv - czf - fone), fzero)
                    return lax.convert_element_type(dz * dz > thr,
                                                    jnp.int32)

                zlo_r = jnp.maximum(czv - 2 + zfail(czv - 2)
                                    + zfail(czv - 1), zero)
                zhi_r = jnp.minimum(czv + 2 - zfail(czv + 2)
                                    - zfail(czv + 1),
                                    jnp.full((L,), gz - 1, jnp.int32))
                cxc = jnp.minimum(jnp.maximum(cx, zero),
                                  jnp.full((L,), gx - 1, jnp.int32))
                cyc = jnp.minimum(jnp.maximum(cy, zero),
                                  jnp.full((L,), gy - 1, jnp.int32))
                bc = cxc * jnp.full((L,), sx, jnp.int32) \
                    + cyc * jnp.full((L,), sy, jnp.int32)
                valid = jnp.logical_and(valid, zhi_r >= zlo_r)
                sidx = bc + jnp.maximum(zlo_r, zero)
                eidx = bc + jnp.minimum(zhi_r,
                                        jnp.full((L,), GMAX - 1,
                                                 jnp.int32)) + 1
                s0v = plsc.load_gather(csv, [sidx])
                e0v = plsc.load_gather(csv, [eidx])
                e0v = jnp.where(valid, e0v, s0v)
                mask_r = e0v > s0v
                plsc.store_compressed(rows_s.at[pl.ds(nr, L)], s0v,
                                      mask=mask_r)
                plsc.store_compressed(rows_e.at[pl.ds(nr, L)], e0v,
                                      mask=mask_r)
                nr = nr + plsc.all_reduce_population_count(mask_r)[0]

            def row(r, cnt):
                s0 = _rd(rows_s, r)
                e = _rd(rows_e, r)

                def cond(carry):
                    return carry[0] < e

                def body(carry):
                    j, c = carry
                    rem = e - j
                    lmask = it < jnp.full((L,), rem, jnp.int32)
                    xv = pv[pl.ds(0 * NP + j, L)]
                    yv = pv[pl.ds(1 * NP + j, L)]
                    zv = pv[pl.ds(2 * NP + j, L)]
                    llv = pv[pl.ds(3 * NP + j, L)]
                    mm = (bqx * xv + bqy * yv) + bqz * zv
                    d2 = (qqv + llv) - (np.float32(2.0) * mm)
                    hit = jnp.logical_and(d2 <= r2v, lmask)
                    pc = plsc.all_reduce_population_count(hit)[0]

                    @pl.when(c < MAX_COLLISIONS)
                    def _():
                        posv = lax.convert_element_type(
                            jnp.full((L,), j, jnp.int32) + it, jnp.float32)
                        plsc.store_compressed(buf.at[pl.ds(c, L)], posv,
                                              mask=hit)
                    return j + L, c + pc
                _, cnt = lax.while_loop(cond, body, (s0, cnt))
                return cnt
            lax.fori_loop(0, nr, row, np.int32(0))

        def group(qc, _):
            def one_q(qs, _):
                q = qc * QG + qs
                do_query(q)

                @pl.loop(0, MAX_COLLISIONS, step=L)
                def _(i):
                    stg[pl.ds(qs * MAX_COLLISIONS + i, L)] = buf[pl.ds(i, L)]
                return 0
            lax.fori_loop(0, QG, one_q, 0)
            pltpu.sync_copy(
                stg,
                nb_hbm.at[pl.ds(
                    b * (M * MAX_COLLISIONS)
                    + (qbase + qc * QG) * MAX_COLLISIONS,
                    QG * MAX_COLLISIONS)])
            return 0
        lax.fori_loop(0, CHUNK // QG, group, 0)

        for gcp in gathers:
            gcp.wait()

        for ch2 in range(4):
            r0 = ch2 * (DC // 4)

            @pl.loop(r0, r0 + DC // 4)
            def _(i, r0=r0):
                h = (_rd(permL, i) & 1) * C
                for kq in range(C // L):
                    v = plsc.load_gather(
                        drows, [jnp.full((L,), i, jnp.int32),
                                jnp.full((L,), h + kq * L, jnp.int32) + it])
                    bounce[pl.ds((i - r0) * C + kq * L, L)] = v

            pltpu.sync_copy(
                bounce,
                datar_hbm.at[pl.ds(b * (N * C) + (dbase + r0) * C,
                                   (DC // 4) * C)])

    return k(qlocs_T, data2, permi, p_arr, cs, mf, mi)


def kernel(locs, data, qlocs):
    B, N, _ = locs.shape
    M = qlocs.shape[1]
    locs_T = jnp.transpose(locs, (0, 2, 1)).reshape(B * NDIM * N)
    qlocs_T = jnp.transpose(qlocs, (0, 2, 1)).reshape(B * NDIM * M)
    C = data.shape[2]
    data2 = data.reshape(B, (N * C) // 128, 128)
    locsr_T, idxs, permi, p_arr, cs, mf, mi = _phase1(locs_T, B)
    neighbors, data_r = _phase2(qlocs_T, data2, permi, p_arr, cs, mf, mi, C)
    locs_r = jnp.transpose(locsr_T.reshape(B, NDIM, N), (0, 2, 1))
    return (locs_r, data_r.reshape(B, N, C), idxs.reshape(B, N),
            neighbors.reshape(B, M, MAX_COLLISIONS))


# speedup vs baseline: 1.5017x; 1.0286x over previous
import dataclasses
import functools

import jax
import jax.numpy as jnp
import numpy as np
from jax import lax
from jax.experimental import pallas as pl
from jax.experimental.pallas import tpu as pltpu
from jax.experimental.pallas import tpu_sc as plsc

NDIM = 3
MAX_COLLISIONS = 128
L = 16
NSUB = 16
GMAX = 12
NCELL = GMAX * GMAX * GMAX
NCELL_PAD = 1792
BUFN = 160

_R2 = np.float32(0.01)
_TEN = np.float32(10.0)
_EPS = np.float32(-1e-6)


def _bf16_round(x):
    u = plsc.bitcast(x, jnp.uint32)
    r = u + np.uint32(0x7FFF) + ((u >> np.uint32(16)) & np.uint32(1))
    r = r & np.uint32(0xFFFF0000)
    return plsc.bitcast(r, jnp.float32)


def _iota16():
    return lax.iota(jnp.int32, L)


def _compiler_params():
    cp = pltpu.CompilerParams()
    if "needs_layout_passes" in pltpu.CompilerParams.__dataclass_fields__:
        cp = dataclasses.replace(cp, needs_layout_passes=False)
    return cp


def _st1(ref, idx, val):
    lane0 = _iota16() == 0
    plsc.store_scatter(ref, [jnp.full((L,), idx, jnp.int32)],
                       jnp.full((L,), val), mask=lane0)


def _rd(ref, idx):
    return ref[pl.ds(idx, L)][0]


def _phase1(locs_T, B):
    N = locs_T.shape[0] // (B * NDIM)
    CHUNK = N // NSUB
    NV = CHUNK // L
    QC = CHUNK // 4
    mesh = plsc.VectorSubcoreMesh(core_axis_name="c", subcore_axis_name="s")

    out_type = (
        jax.ShapeDtypeStruct((B * NDIM * N,), jnp.float32),
        jax.ShapeDtypeStruct((B * N,), jnp.float32),
        jax.ShapeDtypeStruct((B * N,), jnp.int32),
        jax.ShapeDtypeStruct((B * 4 * N,), jnp.float32),
        jax.ShapeDtypeStruct((B * NCELL_PAD,), jnp.int32),
        jax.ShapeDtypeStruct((B * 128,), jnp.float32),
        jax.ShapeDtypeStruct((B * 128,), jnp.int32),
    )
    scratch = [
        pltpu.VMEM((NDIM * CHUNK,), jnp.float32),
        pltpu.VMEM((4 * CHUNK,), jnp.float32),
        pltpu.VMEM((CHUNK,), jnp.int32),
        pltpu.VMEM((QC,), jnp.int32),
        pltpu.VMEM((QC,), jnp.int32),
        pltpu.VMEM((QC,), jnp.int32),
        pltpu.VMEM((QC,), jnp.int32),
        pltpu.VMEM((CHUNK,), jnp.float32),
        pltpu.VMEM((CHUNK,), jnp.int32),
        pltpu.VMEM((NCELL_PAD,), jnp.int32),
        pltpu.VMEM((L,), jnp.float32),
        pltpu.VMEM((NSUB * NCELL_PAD,), jnp.int32),
        pltpu.VMEM((NSUB * NCELL_PAD,), jnp.int32),
        pltpu.VMEM((NCELL_PAD,), jnp.int32),
        pltpu.VMEM((128,), jnp.float32),
        pltpu.VMEM((128,), jnp.int32),
        pltpu.VMEM((NSUB * L,), jnp.float32),
        pltpu.VMEM_SHARED((NSUB * L,), jnp.float32),
        pltpu.VMEM_SHARED((NSUB * NCELL_PAD,), jnp.int32),
        pltpu.VMEM_SHARED((NSUB * NCELL_PAD,), jnp.int32),
        pltpu.VMEM_SHARED((N,), jnp.float32),
        pltpu.VMEM_SHARED((N,), jnp.float32),
        pltpu.VMEM_SHARED((N,), jnp.float32),
        pltpu.VMEM_SHARED((N,), jnp.float32),
        pltpu.VMEM_SHARED((N,), jnp.float32),
        pltpu.VMEM_SHARED((N,), jnp.float32),
        pltpu.VMEM_SHARED((N,), jnp.float32),
        pltpu.VMEM_SHARED((N,), jnp.int32),
        pltpu.SemaphoreType.DMA,
    ]

    @functools.partial(pl.kernel, out_type=out_type, mesh=mesh,
                       scratch_types=scratch,
                       compiler_params=_compiler_params())
    def k(locs_hbm, locsr_hbm, idxs_hbm, permi_hbm, p_hbm,
          cs_hbm, mf_hbm, mi_hbm, xyz, bll, cid, pos0, pos1, pos2, pos3,
          orig, orig_i, hist, mmv, hall, oall, csl, mfs, mis, mm_all,
          sh_mm, sh_hist, sh_off, sh_xr, sh_yr, sh_zr, sh_bx, sh_by,
          sh_bz, sh_ll, sh_oi, sem):
        b = lax.axis_index("c")
        s = lax.axis_index("s")
        base = s * CHUNK
        it = _iota16()

        for a in range(NDIM):
            pltpu.sync_copy(
                locs_hbm.at[pl.ds(b * (NDIM * N) + a * N + base, CHUNK)],
                xyz.at[pl.ds(a * CHUNK, CHUNK)])

        inf = jnp.full((L,), jnp.inf, jnp.float32)
        vec = inf
        for a in range(NDIM):
            def red(i, carry):
                mn, nmx = carry
                v = xyz[pl.ds(a * CHUNK + i * L, L)]
                return jnp.minimum(mn, v), jnp.minimum(nmx, -v)
            mn, nmx = lax.fori_loop(0, NV, red, (inf, inf))
            vec = jnp.where(it == a, jnp.full((L,), jnp.min(mn)), vec)
            vec = jnp.where(it == NDIM + a,
                            jnp.full((L,), jnp.min(nmx)), vec)
        mmv[...] = vec
        pltpu.sync_copy(mmv, sh_mm.at[pl.ds(s * L, L)])
        plsc.subcore_barrier()

        pltpu.sync_copy(sh_mm, mm_all)
        acc = inf
        for t in range(NSUB):
            acc = jnp.minimum(acc, mm_all[pl.ds(t * L, L)])
        mmv[...] = acc
        i3 = jnp.minimum(it, 2)
        mn3 = plsc.load_gather(mmv, [i3])
        nmx3 = plsc.load_gather(mmv, [i3 + 3])
        lower = mn3 + _EPS
        delta = (np.float32(0.0) - nmx3) - lower
        t = delta * _TEN
        ti = lax.convert_element_type(t, jnp.int32)
        tif = lax.convert_element_type(ti, jnp.float32)
        one = jnp.full((L,), 1, jnp.int32)
        zero = jnp.zeros((L,), jnp.int32)
        g = jnp.minimum(ti + jnp.where(t > tif, one, zero) + 1, 96)
        lox = lower[0]
        loy = lower[1]
        loz = lower[2]
        gx = g[0]
        gy = g[1]
        gz = g[2]
        sy = gz
        sx = gy * gz

        gxv = jnp.full((L,), gx - 1, jnp.int32)
        gyv = jnp.full((L,), gy - 1, jnp.int32)
        gzv = jnp.full((L,), gz - 1, jnp.int32)
        sxv = jnp.full((L,), sx, jnp.int32)
        syv = jnp.full((L,), sy, jnp.int32)

        def cells(i, _):
            sl = pl.ds(i * L, L)
            x = xyz[pl.ds(0 * CHUNK + i * L, L)]
            y = xyz[pl.ds(1 * CHUNK + i * L, L)]
            z = xyz[pl.ds(2 * CHUNK + i * L, L)]
            cx = lax.convert_element_type((x - lox) * _TEN, jnp.int32)
            cy = lax.convert_element_type((y - loy) * _TEN, jnp.int32)
            cz = lax.convert_element_type((z - loz) * _TEN, jnp.int32)
            cx = jnp.minimum(jnp.maximum(cx, zero), gxv)
            cy = jnp.minimum(jnp.maximum(cy, zero), gyv)
            cz = jnp.minimum(jnp.maximum(cz, zero), gzv)
            cid[sl] = cx * sxv + cy * syv + cz
            bll[pl.ds(0 * CHUNK + i * L, L)] = _bf16_round(x)
            bll[pl.ds(1 * CHUNK + i * L, L)] = _bf16_round(y)
            bll[pl.ds(2 * CHUNK + i * L, L)] = _bf16_round(z)
            bll[pl.ds(3 * CHUNK + i * L, L)] = (x * x + y * y) + z * z
            oi = jnp.full((L,), base + i * L, jnp.int32) + it
            orig_i[sl] = oi
            orig[sl] = lax.convert_element_type(oi, jnp.float32)
            return 0
        lax.fori_loop(0, NV, cells, 0)

        @pl.loop(0, NCELL_PAD, step=L)
        def _(i):
            hist[pl.ds(i, L)] = zero

        @pl.loop(0, CHUNK, step=L)
        def _(i):
            cv = cid[pl.ds(i, L)]
            for kk in range(L):
                c = cv[kk]
                _st1(hist, c, _rd(hist, c) + 1)

        pltpu.sync_copy(hist, sh_hist.at[pl.ds(s * NCELL_PAD, NCELL_PAD)])
        plsc.subcore_barrier()

        @pl.when(s == 0)
        def _():
            pltpu.sync_copy(sh_hist, hall)

            @pl.loop(NCELL, NCELL_PAD, step=L)
            def _(i):
                csl[pl.ds(i, L)] = jnp.full((L,), N, jnp.int32)

            def scan(kk, bb):
                kv = it * NCELL_PAD + kk
                cnt = plsc.load_gather(hall, [kv])
                incl = plsc.cumsum(cnt)
                excl = incl - cnt
                plsc.store_scatter(oall, [kv],
                                   jnp.full((L,), bb, jnp.int32) + excl)
                _st1(csl, kk, bb)
                return bb + jnp.sum(cnt)
            lax.fori_loop(0, NCELL, scan, np.int32(0))

            pltpu.sync_copy(oall, sh_off)
            pltpu.sync_copy(csl, cs_hbm.at[pl.ds(b * NCELL_PAD, NCELL_PAD)])
            mfs[pl.ds(0, L)] = lower
            mis[pl.ds(0, L)] = jnp.where(
                it == 3, jnp.full((L,), sx, jnp.int32),
                jnp.where(it == 4, jnp.full((L,), sy, jnp.int32), g))
            pltpu.sync_copy(mfs, mf_hbm.at[pl.ds(b * 128, 128)])
            pltpu.sync_copy(mis, mi_hbm.at[pl.ds(b * 128, 128)])
        plsc.subcore_barrier()

        pltpu.sync_copy(sh_off.at[pl.ds(s * NCELL_PAD, NCELL_PAD)], hist)

        for ch, pref in enumerate((pos0, pos1, pos2, pos3)):
            @pl.loop(ch * QC, (ch + 1) * QC, step=L)
            def _(i, pref=pref, ch=ch):
                cv = cid[pl.ds(i, L)]
                for kk in range(L):
                    c = cv[kk]
                    p = _rd(hist, c)
                    _st1(hist, c, p + 1)
                    _st1(pref, i + kk - ch * QC, p)

        copies = []
        for ch, pref in enumerate((pos0, pos1, pos2, pos3)):
            qs = pl.ds(ch * QC, QC)
            copies.append(pltpu.async_copy(
                orig_i.at[qs], sh_oi.at[pref], sem))
            for a, shr in enumerate((sh_xr, sh_yr, sh_zr)):
                copies.append(pltpu.async_copy(
                    xyz.at[pl.ds(a * CHUNK + ch * QC, QC)],
                    shr.at[pref], sem))
            for a, shr in enumerate((sh_bx, sh_by, sh_bz, sh_ll)):
                copies.append(pltpu.async_copy(
                    bll.at[pl.ds(a * CHUNK + ch * QC, QC)],
                    shr.at[pref], sem))
        for cp in copies:
            cp.wait()
        plsc.subcore_barrier()

        csl2 = pl.ds(base, CHUNK)
        for a, shr in enumerate((sh_xr, sh_yr, sh_zr)):
            pltpu.sync_copy(
                shr.at[csl2],
                locsr_hbm.at[pl.ds(b * (NDIM * N) + a * N + base, CHUNK)])
        for a, shr in enumerate((sh_bx, sh_by, sh_bz, sh_ll)):
            pltpu.sync_copy(
                shr.at[csl2],
                p_hbm.at[pl.ds(b * (4 * N) + a * N + base, CHUNK)])
        pltpu.sync_copy(sh_oi.at[csl2],
                        permi_hbm.at[pl.ds(b * N + base, CHUNK)])
        pltpu.sync_copy(sh_oi.at[csl2], orig_i)

        @pl.loop(0, NV, step=1)
        def _(i):
            orig[pl.ds(i * L, L)] = lax.convert_element_type(
                orig_i[pl.ds(i * L, L)], jnp.float32)
        pltpu.sync_copy(orig, idxs_hbm.at[pl.ds(b * N + base, CHUNK)])

    return k(locs_T)


def _phase2(qlocs_T, data2, permi, p_arr, cs, mf, mi, C):
    B = data2.shape[0]
    M = qlocs_T.shape[0] // (B * NDIM)
    N = p_arr.shape[0] // (B * 4)
    NP = N + L
    CHUNK = M // NSUB
    CP = CHUNK + L
    DC = N // NSUB
    QC = DC // 4
    QG = 32
    mesh = plsc.VectorSubcoreMesh(core_axis_name="c", subcore_axis_name="s")

    out_type = (
        jax.ShapeDtypeStruct((B * M * MAX_COLLISIONS,), jnp.float32),
        jax.ShapeDtypeStruct((B * N * C,), jnp.float32),
    )
    scratch = [
        pltpu.VMEM((4 * NP,), jnp.float32),
        pltpu.VMEM((NCELL_PAD,), jnp.int32),
        pltpu.VMEM((NDIM * CHUNK,), jnp.float32),
        pltpu.VMEM((5 * CP,), jnp.float32),
        pltpu.VMEM((NDIM * CP,), jnp.int32),
        pltpu.VMEM((NDIM * CP,), jnp.float32),
        pltpu.VMEM((4 * L,), jnp.int32),
        pltpu.VMEM((4 * L,), jnp.int32),
        pltpu.VMEM((BUFN,), jnp.float32),
        pltpu.VMEM((QG * MAX_COLLISIONS + L,), jnp.float32),
        pltpu.VMEM((128,), jnp.float32),
        pltpu.VMEM((128,), jnp.int32),
        pltpu.VMEM((QC,), jnp.int32),
        pltpu.VMEM((QC,), jnp.int32),
        pltpu.VMEM((QC,), jnp.int32),
        pltpu.VMEM((QC,), jnp.int32),
        pltpu.VMEM((DC + L,), jnp.int32),
        pltpu.VMEM((DC, 128), jnp.float32),
        pltpu.VMEM((128 * 64,), jnp.float32),
        pltpu.SemaphoreType.DMA,
        pltpu.SemaphoreType.DMA,
    ]

    @functools.partial(pl.kernel, out_type=out_type, mesh=mesh,
                       scratch_types=scratch,
                       compiler_params=_compiler_params())
    def k(q_hbm, data_hbm, permi_hbm, p_hbm, cs_hbm, mf_hbm, mi_hbm,
          nb_hbm, datar_hbm, pv, csv, qv, bq, cq, uq, rows_s, rows_e,
          buf, stg, mfs, mis, gi0, gi1, gi2, gi3, permL, drows, bounce,
          sem, gsem):
        b = lax.axis_index("c")
        s = lax.axis_index("s")
        qbase = s * CHUNK
        dbase = s * DC
        it = _iota16()

        pltpu.sync_copy(permi_hbm.at[pl.ds(b * N + dbase, DC)],
                        permL.at[pl.ds(0, DC)])
        girefs = (gi0, gi1, gi2, gi3)
        for ch, gir in enumerate(girefs):
            @pl.loop(0, QC, step=L)
            def _(i, gir=gir, ch=ch):
                gir[pl.ds(i, L)] = permL[pl.ds(ch * QC + i, L)] >> 1
        gathers = [
            pltpu.async_copy(data_hbm.at[b].at[gir],
                             drows.at[pl.ds(ch * QC, QC), :], gsem)
            for ch, gir in enumerate(girefs)
        ]

        for a in range(4):
            pltpu.sync_copy(p_hbm.at[pl.ds(b * (4 * N) + a * N, N)],
                            pv.at[pl.ds(a * NP, N)])
        pltpu.sync_copy(cs_hbm.at[pl.ds(b * NCELL_PAD, NCELL_PAD)], csv)
        pltpu.sync_copy(mf_hbm.at[pl.ds(b * 128, 128)], mfs)
        pltpu.sync_copy(mi_hbm.at[pl.ds(b * 128, 128)], mis)
        for a in range(NDIM):
            pltpu.sync_copy(
                q_hbm.at[pl.ds(b * (NDIM * M) + a * M + qbase, CHUNK)],
                qv.at[pl.ds(a * CHUNK, CHUNK)])

        mv = mfs[pl.ds(0, L)]
        miv = mis[pl.ds(0, L)]
        lox = mv[0]
        loy = mv[1]
        loz = mv[2]
        gx = miv[0]
        gy = miv[1]
        gz = miv[2]
        sx = miv[3]
        sy = miv[4]

        one = jnp.full((L,), 1, jnp.int32)
        zero = jnp.zeros((L,), jnp.int32)

        def prep(i, _):
            x = qv[pl.ds(0 * CHUNK + i * L, L)]
            y = qv[pl.ds(1 * CHUNK + i * L, L)]
            z = qv[pl.ds(2 * CHUNK + i * L, L)]
            bx = _bf16_round(x)
            by = _bf16_round(y)
            bz = _bf16_round(z)
            bq[pl.ds(0 * CP + i * L, L)] = bx
            bq[pl.ds(1 * CP + i * L, L)] = by
            bq[pl.ds(2 * CP + i * L, L)] = bz
            bq[pl.ds(3 * CP + i * L, L)] = (x * x + y * y) + z * z
            c8 = np.float32(0.00396)
            fone_ = jnp.full((L,), np.float32(1.0), jnp.float32)
            r24 = np.float32(0.24)
            s2 = (jnp.minimum(fone_, x + r24) * (jnp.abs(x - bx) + bx * c8)
                  + jnp.minimum(fone_, y + r24) * (jnp.abs(y - by) + by * c8)
                  + jnp.minimum(fone_, z + r24) * (jnp.abs(z - bz) + bz * c8))
            bq[pl.ds(4 * CP + i * L, L)] = (jnp.full((L,), np.float32(1.01))
                                            + np.float32(200.8) * s2)
            for a, w in ((0, x - lox), (1, y - loy), (2, z - loz)):
                t = w * _TEN
                ti = lax.convert_element_type(t, jnp.int32)
                tif = lax.convert_element_type(ti, jnp.float32)
                cq[pl.ds(a * CP + i * L, L)] = ti - jnp.where(t < tif, one,
                                                              zero)
                uq[pl.ds(a * CP + i * L, L)] = t
            return 0
        lax.fori_loop(0, CHUNK // L, prep, 0)

        neg1 = jnp.full((L,), -1.0, jnp.float32)
        r2v = jnp.full((L,), _R2, jnp.float32)

        def do_query(q, obase):
            i3m = jnp.minimum(it, 2)
            qfv = jnp.full((L,), q, jnp.int32)
            qb4 = plsc.load_gather(bq, [qfv + CP * jnp.minimum(it, 4)])
            cq3 = plsc.load_gather(cq, [qfv + CP * i3m])
            uq3 = plsc.load_gather(uq, [qfv + CP * i3m])
            bqx = jnp.full((L,), qb4[0], jnp.float32)
            bqy = jnp.full((L,), qb4[1], jnp.float32)
            bqz = jnp.full((L,), qb4[2], jnp.float32)
            qqv = jnp.full((L,), qb4[3], jnp.float32)
            cqx = cq3[0]
            cqy = cq3[1]
            cqz = cq3[2]

            @pl.loop(0, MAX_COLLISIONS, step=L)
            def _(i):
                stg[pl.ds(obase + i, L)] = neg1

            tc2 = jnp.full((L,), qb4[4], jnp.float32)
            uxv = jnp.full((L,), uq3[0], jnp.float32)
            uyv = jnp.full((L,), uq3[1], jnp.float32)
            uzv = jnp.full((L,), uq3[2], jnp.float32)
            czv = jnp.full((L,), cqz, jnp.int32)
            zoks = jnp.logical_and(cqz + 2 >= 0, cqz - 2 <= gz - 1)
            fone = jnp.full((L,), np.float32(1.0), jnp.float32)
            fzero = jnp.zeros((L,), jnp.float32)
            nr = cqz * 0

            for h in range(2):
                rr = it + h * L
                rx = rr // 5 - 2
                ry = rr % 5 - 2
                cx = jnp.full((L,), cqx, jnp.int32) + rx
                cy = jnp.full((L,), cqy, jnp.int32) + ry
                cxf = lax.convert_element_type(cx, jnp.float32)
                cyf = lax.convert_element_type(cy, jnp.float32)
                dxm = jnp.maximum(
                    jnp.maximum(cxf - uxv, uxv - cxf - fone), fzero)
                dym = jnp.maximum(
                    jnp.maximum(cyf - uyv, uyv - cyf - fone), fzero)
                rxy2 = dxm * dxm + dym * dym
                valid = jnp.logical_and(
                    jnp.logical_and(rr < 25, rxy2 <= tc2),
                    jnp.logical_and(
                        jnp.logical_and(cx >= 0, cx < jnp.full((L,), gx,
                                                            jnp.int32)),
                        jnp.logical_and(cy >= 0, cy < jnp.full((L,), gy,
                                                            jnp.int32))))
                valid = jnp.logical_and(valid, zoks)
                thr = tc2 - rxy2

                def zfail(zc):
                    czf = lax.convert_element_type(zc, jnp.float32)
                    dz = jnp.maximum(
                        jnp.maximum(czf - uzv, uzv - czf - fone), fzero)
                    return lax.convert_element_type(dz * dz > thr,
                                                    jnp.int32)

                zlo_r = jnp.maximum(czv - 2 + zfail(czv - 2)
                                    + zfail(czv - 1), zero)
                zhi_r = jnp.minimum(czv + 2 - zfail(czv + 2)
                                    - zfail(czv + 1),
                                    jnp.full((L,), gz - 1, jnp.int32))
                cxc = jnp.minimum(jnp.maximum(cx, zero),
                                  jnp.full((L,), gx - 1, jnp.int32))
                cyc = jnp.minimum(jnp.maximum(cy, zero),
                                  jnp.full((L,), gy - 1, jnp.int32))
                bc = cxc * jnp.full((L,), sx, jnp.int32) \
                    + cyc * jnp.full((L,), sy, jnp.int32)
                valid = jnp.logical_and(valid, zhi_r >= zlo_r)
                sidx = bc + jnp.maximum(zlo_r, zero)
                eidx = bc + jnp.minimum(zhi_r,
                                        jnp.full((L,), GMAX - 1,
                                                 jnp.int32)) + 1
                s0v = plsc.load_gather(csv, [sidx])
                e0v = plsc.load_gather(csv, [eidx])
                e0v = jnp.where(valid, e0v, s0v)
                mask_r = e0v > s0v
                plsc.store_compressed(rows_s.at[pl.ds(nr, L)], s0v,
                                      mask=mask_r)
                plsc.store_compressed(rows_e.at[pl.ds(nr, L)], e0v,
                                      mask=mask_r)
                nr = nr + plsc.all_reduce_population_count(mask_r)[0]

            def row(r, cnt):
                s0 = _rd(rows_s, r)
                e = _rd(rows_e, r)

                def cond(carry):
                    return carry[0] < e

                def body(carry):
                    j, c = carry
                    rem = e - j
                    lmask = it < jnp.full((L,), rem, jnp.int32)
                    xv = pv[pl.ds(0 * NP + j, L)]
                    yv = pv[pl.ds(1 * NP + j, L)]
                    zv = pv[pl.ds(2 * NP + j, L)]
                    llv = pv[pl.ds(3 * NP + j, L)]
                    mm = (bqx * xv + bqy * yv) + bqz * zv
                    d2 = (qqv + llv) - (np.float32(2.0) * mm)
                    hit = jnp.logical_and(d2 <= r2v, lmask)
                    pc = plsc.all_reduce_population_count(hit)[0]

                    @pl.when(c < MAX_COLLISIONS)
                    def _():
                        posv = lax.convert_element_type(
                            jnp.full((L,), j, jnp.int32) + it, jnp.float32)
                        plsc.store_compressed(
                            stg.at[pl.ds(obase + c, L)], posv, mask=hit)
                    return j + L, c + pc
                _, cnt = lax.while_loop(cond, body, (s0, cnt))
                return cnt
            lax.fori_loop(0, nr, row, np.int32(0))

        def group(qc, _):
            def one_q(qs, _):
                do_query(qc * QG + qs, qs * MAX_COLLISIONS)
                return 0
            lax.fori_loop(0, QG, one_q, 0)
            pltpu.sync_copy(
                stg.at[pl.ds(0, QG * MAX_COLLISIONS)],
                nb_hbm.at[pl.ds(
                    b * (M * MAX_COLLISIONS)
                    + (qbase + qc * QG) * MAX_COLLISIONS,
                    QG * MAX_COLLISIONS)])
            return 0
        lax.fori_loop(0, CHUNK // QG, group, 0)

        for gcp in gathers:
            gcp.wait()

        for ch2 in range(4):
            r0 = ch2 * (DC // 4)

            @pl.loop(r0, r0 + DC // 4)
            def _(i, r0=r0):
                h = (_rd(permL, i) & 1) * C
                for kq in range(C // L):
                    v = plsc.load_gather(
                        drows, [jnp.full((L,), i, jnp.int32),
                                jnp.full((L,), h + kq * L, jnp.int32) + it])
                    bounce[pl.ds((i - r0) * C + kq * L, L)] = v

            pltpu.sync_copy(
                bounce,
                datar_hbm.at[pl.ds(b * (N * C) + (dbase + r0) * C,
                                   (DC // 4) * C)])

    return k(qlocs_T, data2, permi, p_arr, cs, mf, mi)


def kernel(locs, data, qlocs):
    B, N, _ = locs.shape
    M = qlocs.shape[1]
    locs_T = jnp.transpose(locs, (0, 2, 1)).reshape(B * NDIM * N)
    qlocs_T = jnp.transpose(qlocs, (0, 2, 1)).reshape(B * NDIM * M)
    C = data.shape[2]
    data2 = data.reshape(B, (N * C) // 128, 128)
    locsr_T, idxs, permi, p_arr, cs, mf, mi = _phase1(locs_T, B)
    neighbors, data_r = _phase2(qlocs_T, data2, permi, p_arr, cs, mf, mi, C)
    locs_r = jnp.transpose(locsr_T.reshape(B, NDIM, N), (0, 2, 1))
    return (locs_r, data_r.reshape(B, N, C), idxs.reshape(B, N),
            neighbors.reshape(B, M, MAX_COLLISIONS))


# speedup vs baseline: 1.6146x; 1.0752x over previous
import dataclasses
import functools

import jax
import jax.numpy as jnp
import numpy as np
from jax import lax
from jax.experimental import pallas as pl
from jax.experimental.pallas import tpu as pltpu
from jax.experimental.pallas import tpu_sc as plsc

NDIM = 3
MAX_COLLISIONS = 128
L = 16
NSUB = 16
GMAX = 12
NCELL = GMAX * GMAX * GMAX
NCELL_PAD = 1792
BUFN = 160

_R2 = np.float32(0.01)
_TEN = np.float32(10.0)
_EPS = np.float32(-1e-6)


def _bf16_round(x):
    u = plsc.bitcast(x, jnp.uint32)
    r = u + np.uint32(0x7FFF) + ((u >> np.uint32(16)) & np.uint32(1))
    r = r & np.uint32(0xFFFF0000)
    return plsc.bitcast(r, jnp.float32)


def _iota16():
    return lax.iota(jnp.int32, L)


def _compiler_params():
    cp = pltpu.CompilerParams()
    if "needs_layout_passes" in pltpu.CompilerParams.__dataclass_fields__:
        cp = dataclasses.replace(cp, needs_layout_passes=False)
    return cp


def _st1(ref, idx, val):
    lane0 = _iota16() == 0
    plsc.store_scatter(ref, [jnp.full((L,), idx, jnp.int32)],
                       jnp.full((L,), val), mask=lane0)


def _rd(ref, idx):
    return ref[pl.ds(idx, L)][0]


def _phase1(locs_T, B):
    N = locs_T.shape[0] // (B * NDIM)
    CHUNK = N // NSUB
    NV = CHUNK // L
    QC = CHUNK // 4
    mesh = plsc.VectorSubcoreMesh(core_axis_name="c", subcore_axis_name="s")

    out_type = (
        jax.ShapeDtypeStruct((B * NDIM * N,), jnp.float32),
        jax.ShapeDtypeStruct((B * N,), jnp.float32),
        jax.ShapeDtypeStruct((B * N,), jnp.int32),
        jax.ShapeDtypeStruct((B * 4 * N,), jnp.float32),
        jax.ShapeDtypeStruct((B * NCELL_PAD,), jnp.int32),
        jax.ShapeDtypeStruct((B * 128,), jnp.float32),
        jax.ShapeDtypeStruct((B * 128,), jnp.int32),
    )
    scratch = [
        pltpu.VMEM((NDIM * CHUNK,), jnp.float32),
        pltpu.VMEM((4 * CHUNK,), jnp.float32),
        pltpu.VMEM((CHUNK,), jnp.int32),
        pltpu.VMEM((QC,), jnp.int32),
        pltpu.VMEM((QC,), jnp.int32),
        pltpu.VMEM((QC,), jnp.int32),
        pltpu.VMEM((QC,), jnp.int32),
        pltpu.VMEM((CHUNK,), jnp.float32),
        pltpu.VMEM((CHUNK,), jnp.int32),
        pltpu.VMEM((NCELL_PAD,), jnp.int32),
        pltpu.VMEM((L,), jnp.float32),
        pltpu.VMEM((NSUB * NCELL_PAD,), jnp.int32),
        pltpu.VMEM((NSUB * NCELL_PAD,), jnp.int32),
        pltpu.VMEM((NCELL_PAD,), jnp.int32),
        pltpu.VMEM((128,), jnp.float32),
        pltpu.VMEM((128,), jnp.int32),
        pltpu.VMEM((NSUB * L,), jnp.float32),
        pltpu.VMEM((2 * L,), jnp.int32),
        pltpu.VMEM_SHARED((NSUB * 8,), jnp.int32),
        pltpu.VMEM_SHARED((NSUB * L,), jnp.float32),
        pltpu.VMEM_SHARED((NSUB * NCELL_PAD,), jnp.int32),
        pltpu.VMEM_SHARED((NSUB * NCELL_PAD,), jnp.int32),
        pltpu.VMEM_SHARED((N,), jnp.float32),
        pltpu.VMEM_SHARED((N,), jnp.float32),
        pltpu.VMEM_SHARED((N,), jnp.float32),
        pltpu.VMEM_SHARED((N,), jnp.float32),
        pltpu.VMEM_SHARED((N,), jnp.float32),
        pltpu.VMEM_SHARED((N,), jnp.float32),
        pltpu.VMEM_SHARED((N,), jnp.float32),
        pltpu.VMEM_SHARED((N,), jnp.int32),
        pltpu.SemaphoreType.DMA,
    ]

    @functools.partial(pl.kernel, out_type=out_type, mesh=mesh,
                       scratch_types=scratch,
                       compiler_params=_compiler_params())
    def k(locs_hbm, locsr_hbm, idxs_hbm, permi_hbm, p_hbm,
          cs_hbm, mf_hbm, mi_hbm, xyz, bll, cid, pos0, pos1, pos2, pos3,
          orig, orig_i, hist, mmv, hall, oall, csl, mfs, mis, mm_all,
          tots, sh_tot, sh_mm, sh_hist, sh_off, sh_xr, sh_yr, sh_zr,
          sh_bx, sh_by, sh_bz, sh_ll, sh_oi, sem):
        b = lax.axis_index("c")
        s = lax.axis_index("s")
        base = s * CHUNK
        it = _iota16()

        for a in range(NDIM):
            pltpu.sync_copy(
                locs_hbm.at[pl.ds(b * (NDIM * N) + a * N + base, CHUNK)],
                xyz.at[pl.ds(a * CHUNK, CHUNK)])

        inf = jnp.full((L,), jnp.inf, jnp.float32)
        vec = inf
        for a in range(NDIM):
            def red(i, carry):
                mn, nmx = carry
                v = xyz[pl.ds(a * CHUNK + i * L, L)]
                return jnp.minimum(mn, v), jnp.minimum(nmx, -v)
            mn, nmx = lax.fori_loop(0, NV, red, (inf, inf))
            vec = jnp.where(it == a, jnp.full((L,), jnp.min(mn)), vec)
            vec = jnp.where(it == NDIM + a,
                            jnp.full((L,), jnp.min(nmx)), vec)
        mmv[...] = vec
        pltpu.sync_copy(mmv, sh_mm.at[pl.ds(s * L, L)])
        plsc.subcore_barrier()

        pltpu.sync_copy(sh_mm, mm_all)
        acc = inf
        for t in range(NSUB):
            acc = jnp.minimum(acc, mm_all[pl.ds(t * L, L)])
        mmv[...] = acc
        i3 = jnp.minimum(it, 2)
        mn3 = plsc.load_gather(mmv, [i3])
        nmx3 = plsc.load_gather(mmv, [i3 + 3])
        lower = mn3 + _EPS
        delta = (np.float32(0.0) - nmx3) - lower
        t = delta * _TEN
        ti = lax.convert_element_type(t, jnp.int32)
        tif = lax.convert_element_type(ti, jnp.float32)
        one = jnp.full((L,), 1, jnp.int32)
        zero = jnp.zeros((L,), jnp.int32)
        g = jnp.minimum(ti + jnp.where(t > tif, one, zero) + 1, 96)
        lox = lower[0]
        loy = lower[1]
        loz = lower[2]
        gx = g[0]
        gy = g[1]
        gz = g[2]
        sy = gz
        sx = gy * gz

        gxv = jnp.full((L,), gx - 1, jnp.int32)
        gyv = jnp.full((L,), gy - 1, jnp.int32)
        gzv = jnp.full((L,), gz - 1, jnp.int32)
        sxv = jnp.full((L,), sx, jnp.int32)
        syv = jnp.full((L,), sy, jnp.int32)

        def cells(i, _):
            sl = pl.ds(i * L, L)
            x = xyz[pl.ds(0 * CHUNK + i * L, L)]
            y = xyz[pl.ds(1 * CHUNK + i * L, L)]
            z = xyz[pl.ds(2 * CHUNK + i * L, L)]
            cx = lax.convert_element_type((x - lox) * _TEN, jnp.int32)
            cy = lax.convert_element_type((y - loy) * _TEN, jnp.int32)
            cz = lax.convert_element_type((z - loz) * _TEN, jnp.int32)
            cx = jnp.minimum(jnp.maximum(cx, zero), gxv)
            cy = jnp.minimum(jnp.maximum(cy, zero), gyv)
            cz = jnp.minimum(jnp.maximum(cz, zero), gzv)
            cid[sl] = cx * sxv + cy * syv + cz
            bll[pl.ds(0 * CHUNK + i * L, L)] = _bf16_round(x)
            bll[pl.ds(1 * CHUNK + i * L, L)] = _bf16_round(y)
            bll[pl.ds(2 * CHUNK + i * L, L)] = _bf16_round(z)
            bll[pl.ds(3 * CHUNK + i * L, L)] = (x * x + y * y) + z * z
            oi = jnp.full((L,), base + i * L, jnp.int32) + it
            orig_i[sl] = oi
            orig[sl] = lax.convert_element_type(oi, jnp.float32)
            return 0
        lax.fori_loop(0, NV, cells, 0)

        @pl.loop(0, NCELL_PAD, step=L)
        def _(i):
            hist[pl.ds(i, L)] = zero

        @pl.loop(0, CHUNK, step=L)
        def _(i):
            cv = cid[pl.ds(i, L)]
            for kk in range(L):
                c = cv[kk]
                _st1(hist, c, _rd(hist, c) + 1)

        pltpu.sync_copy(hist, sh_hist.at[pl.ds(s * NCELL_PAD, NCELL_PAD)])
        plsc.subcore_barrier()

        KC = NCELL_PAD // NSUB
        for tt in range(NSUB):
            pltpu.sync_copy(
                sh_hist.at[pl.ds(tt * NCELL_PAD + s * KC, KC)],
                hall.at[pl.ds(tt * KC, KC)])

        def scan(kk, bb):
            kv = it * KC + kk
            cnt = plsc.load_gather(hall, [kv])
            incl = plsc.cumsum(cnt)
            excl = incl - cnt
            plsc.store_scatter(oall, [kv],
                               jnp.full((L,), bb, jnp.int32) + excl)
            _st1(csl, kk, bb)
            return bb + jnp.sum(cnt)
        ctot = lax.fori_loop(0, KC, scan, b * 0)

        tots[pl.ds(0, L)] = jnp.full((L,), ctot, jnp.int32)
        pltpu.sync_copy(tots.at[pl.ds(0, 8)], sh_tot.at[pl.ds(s * 8, 8)])
        plsc.subcore_barrier()
        pltpu.sync_copy(sh_tot, hall.at[pl.ds(0, NSUB * 8)])
        totv = plsc.load_gather(hall, [it * 8])
        exct = plsc.cumsum(totv) - totv
        tots[pl.ds(L, L)] = exct
        bse = _rd(tots, L + s)

        @pl.loop(0, KC, step=L)
        def _(i):
            csl[pl.ds(i, L)] = csl[pl.ds(i, L)] + jnp.full((L,), bse,
                                                           jnp.int32)

        @pl.loop(0, NSUB * KC, step=L)
        def _(i):
            oall[pl.ds(i, L)] = oall[pl.ds(i, L)] + jnp.full((L,), bse,
                                                             jnp.int32)

        for tt in range(NSUB):
            pltpu.sync_copy(
                oall.at[pl.ds(tt * KC, KC)],
                sh_off.at[pl.ds(tt * NCELL_PAD + s * KC, KC)])
        pltpu.sync_copy(
            csl.at[pl.ds(0, KC)],
            cs_hbm.at[pl.ds(b * NCELL_PAD + s * KC, KC)])

        @pl.when(s == 0)
        def _():
            mfs[pl.ds(0, L)] = lower
            mis[pl.ds(0, L)] = jnp.where(
                it == 3, jnp.full((L,), sx, jnp.int32),
                jnp.where(it == 4, jnp.full((L,), sy, jnp.int32), g))
            pltpu.sync_copy(mfs, mf_hbm.at[pl.ds(b * 128, 128)])
            pltpu.sync_copy(mis, mi_hbm.at[pl.ds(b * 128, 128)])
        plsc.subcore_barrier()

        pltpu.sync_copy(sh_off.at[pl.ds(s * NCELL_PAD, NCELL_PAD)], hist)

        for ch, pref in enumerate((pos0, pos1, pos2, pos3)):
            @pl.loop(ch * QC, (ch + 1) * QC, step=L)
            def _(i, pref=pref, ch=ch):
                cv = cid[pl.ds(i, L)]
                for kk in range(L):
                    c = cv[kk]
                    p = _rd(hist, c)
                    _st1(hist, c, p + 1)
                    _st1(pref, i + kk - ch * QC, p)

        copies = []
        for ch, pref in enumerate((pos0, pos1, pos2, pos3)):
            qs = pl.ds(ch * QC, QC)
            copies.append(pltpu.async_copy(
                orig_i.at[qs], sh_oi.at[pref], sem))
            for a, shr in enumerate((sh_xr, sh_yr, sh_zr)):
                copies.append(pltpu.async_copy(
                    xyz.at[pl.ds(a * CHUNK + ch * QC, QC)],
                    shr.at[pref], sem))
            for a, shr in enumerate((sh_bx, sh_by, sh_bz, sh_ll)):
                copies.append(pltpu.async_copy(
                    bll.at[pl.ds(a * CHUNK + ch * QC, QC)],
                    shr.at[pref], sem))
        for cp in copies:
            cp.wait()
        plsc.subcore_barrier()

        csl2 = pl.ds(base, CHUNK)
        for a, shr in enumerate((sh_xr, sh_yr, sh_zr)):
            pltpu.sync_copy(
                shr.at[csl2],
                locsr_hbm.at[pl.ds(b * (NDIM * N) + a * N + base, CHUNK)])
        for a, shr in enumerate((sh_bx, sh_by, sh_bz, sh_ll)):
            pltpu.sync_copy(
                shr.at[csl2],
                p_hbm.at[pl.ds(b * (4 * N) + a * N + base, CHUNK)])
        pltpu.sync_copy(sh_oi.at[csl2],
                        permi_hbm.at[pl.ds(b * N + base, CHUNK)])
        pltpu.sync_copy(sh_oi.at[csl2], orig_i)

        @pl.loop(0, NV, step=1)
        def _(i):
            orig[pl.ds(i * L, L)] = lax.convert_element_type(
                orig_i[pl.ds(i * L, L)], jnp.float32)
        pltpu.sync_copy(orig, idxs_hbm.at[pl.ds(b * N + base, CHUNK)])

    return k(locs_T)


def _phase2(qlocs_T, data2, permi, p_arr, cs, mf, mi, C):
    B = data2.shape[0]
    M = qlocs_T.shape[0] // (B * NDIM)
    N = p_arr.shape[0] // (B * 4)
    NP = N + L
    CHUNK = M // NSUB
    CP = CHUNK + L
    DC = N // NSUB
    QC = DC // 4
    QG = 32
    mesh = plsc.VectorSubcoreMesh(core_axis_name="c", subcore_axis_name="s")

    out_type = (
        jax.ShapeDtypeStruct((B * M * MAX_COLLISIONS,), jnp.float32),
        jax.ShapeDtypeStruct((B * N * C,), jnp.float32),
    )
    scratch = [
        pltpu.VMEM((4 * NP,), jnp.float32),
        pltpu.VMEM((NCELL_PAD,), jnp.int32),
        pltpu.VMEM((NDIM * CHUNK,), jnp.float32),
        pltpu.VMEM((5 * CP,), jnp.float32),
        pltpu.VMEM((NDIM * CP,), jnp.int32),
        pltpu.VMEM((NDIM * CP,), jnp.float32),
        pltpu.VMEM((4 * L,), jnp.int32),
        pltpu.VMEM((4 * L,), jnp.int32),
        pltpu.VMEM((BUFN,), jnp.float32),
        pltpu.VMEM((QG * MAX_COLLISIONS + L,), jnp.float32),
        pltpu.VMEM((128,), jnp.float32),
        pltpu.VMEM((128,), jnp.int32),
        pltpu.VMEM((QC,), jnp.int32),
        pltpu.VMEM((QC,), jnp.int32),
        pltpu.VMEM((QC,), jnp.int32),
        pltpu.VMEM((QC,), jnp.int32),
        pltpu.VMEM((DC + L,), jnp.int32),
        pltpu.VMEM((DC, 128), jnp.float32),
        pltpu.VMEM((128 * 64,), jnp.float32),
        pltpu.SemaphoreType.DMA,
        pltpu.SemaphoreType.DMA,
    ]

    @functools.partial(pl.kernel, out_type=out_type, mesh=mesh,
                       scratch_types=scratch,
                       compiler_params=_compiler_params())
    def k(q_hbm, data_hbm, permi_hbm, p_hbm, cs_hbm, mf_hbm, mi_hbm,
          nb_hbm, datar_hbm, pv, csv, qv, bq, cq, uq, rows_s, rows_e,
          buf, stg, mfs, mis, gi0, gi1, gi2, gi3, permL, drows, bounce,
          sem, gsem):
        b = lax.axis_index("c")
        s = lax.axis_index("s")
        qbase = s * CHUNK
        dbase = s * DC
        it = _iota16()

        pltpu.sync_copy(permi_hbm.at[pl.ds(b * N + dbase, DC)],
                        permL.at[pl.ds(0, DC)])
        girefs = (gi0, gi1, gi2, gi3)
        for ch, gir in enumerate(girefs):
            @pl.loop(0, QC, step=L)
            def _(i, gir=gir, ch=ch):
                gir[pl.ds(i, L)] = permL[pl.ds(ch * QC + i, L)] >> 1
        gathers = [
            pltpu.async_copy(data_hbm.at[b].at[gir],
                             drows.at[pl.ds(ch * QC, QC), :], gsem)
            for ch, gir in enumerate(girefs)
        ]

        for a in range(4):
            pltpu.sync_copy(p_hbm.at[pl.ds(b * (4 * N) + a * N, N)],
                            pv.at[pl.ds(a * NP, N)])
        pltpu.sync_copy(cs_hbm.at[pl.ds(b * NCELL_PAD, NCELL_PAD)], csv)
        pltpu.sync_copy(mf_hbm.at[pl.ds(b * 128, 128)], mfs)
        pltpu.sync_copy(mi_hbm.at[pl.ds(b * 128, 128)], mis)
        for a in range(NDIM):
            pltpu.sync_copy(
                q_hbm.at[pl.ds(b * (NDIM * M) + a * M + qbase, CHUNK)],
                qv.at[pl.ds(a * CHUNK, CHUNK)])

        mv = mfs[pl.ds(0, L)]
        miv = mis[pl.ds(0, L)]
        lox = mv[0]
        loy = mv[1]
        loz = mv[2]
        gx = miv[0]
        gy = miv[1]
        gz = miv[2]
        sx = miv[3]
        sy = miv[4]

        one = jnp.full((L,), 1, jnp.int32)
        zero = jnp.zeros((L,), jnp.int32)

        def prep(i, _):
            x = qv[pl.ds(0 * CHUNK + i * L, L)]
            y = qv[pl.ds(1 * CHUNK + i * L, L)]
            z = qv[pl.ds(2 * CHUNK + i * L, L)]
            bx = _bf16_round(x)
            by = _bf16_round(y)
            bz = _bf16_round(z)
            bq[pl.ds(0 * CP + i * L, L)] = bx
            bq[pl.ds(1 * CP + i * L, L)] = by
            bq[pl.ds(2 * CP + i * L, L)] = bz
            bq[pl.ds(3 * CP + i * L, L)] = (x * x + y * y) + z * z
            c8 = np.float32(0.00396)
            fone_ = jnp.full((L,), np.float32(1.0), jnp.float32)
            r24 = np.float32(0.24)
            s2 = (jnp.minimum(fone_, x + r24) * (jnp.abs(x - bx) + bx * c8)
                  + jnp.minimum(fone_, y + r24) * (jnp.abs(y - by) + by * c8)
                  + jnp.minimum(fone_, z + r24) * (jnp.abs(z - bz) + bz * c8))
            bq[pl.ds(4 * CP + i * L, L)] = (jnp.full((L,), np.float32(1.01))
                                            + np.float32(200.8) * s2)
            for a, w in ((0, x - lox), (1, y - loy), (2, z - loz)):
                t = w * _TEN
                ti = lax.convert_element_type(t, jnp.int32)
                tif = lax.convert_element_type(ti, jnp.float32)
                cq[pl.ds(a * CP + i * L, L)] = ti - jnp.where(t < tif, one,
                                                              zero)
                uq[pl.ds(a * CP + i * L, L)] = t
            return 0
        lax.fori_loop(0, CHUNK // L, prep, 0)

        neg1 = jnp.full((L,), -1.0, jnp.float32)
        r2v = jnp.full((L,), _R2, jnp.float32)

        def do_query(q, obase):
            i3m = jnp.minimum(it, 2)
            qfv = jnp.full((L,), q, jnp.int32)
            qb4 = plsc.load_gather(bq, [qfv + CP * jnp.minimum(it, 4)])
            cq3 = plsc.load_gather(cq, [qfv + CP * i3m])
            uq3 = plsc.load_gather(uq, [qfv + CP * i3m])
            bqx = jnp.full((L,), qb4[0], jnp.float32)
            bqy = jnp.full((L,), qb4[1], jnp.float32)
            bqz = jnp.full((L,), qb4[2], jnp.float32)
            qqv = jnp.full((L,), qb4[3], jnp.float32)
            cqx = cq3[0]
            cqy = cq3[1]
            cqz = cq3[2]

            @pl.loop(0, MAX_COLLISIONS, step=L)
            def _(i):
                stg[pl.ds(obase + i, L)] = neg1

            tc2 = jnp.full((L,), qb4[4], jnp.float32)
            uxv = jnp.full((L,), uq3[0], jnp.float32)
            uyv = jnp.full((L,), uq3[1], jnp.float32)
            uzv = jnp.full((L,), uq3[2], jnp.float32)
            czv = jnp.full((L,), cqz, jnp.int32)
            zoks = jnp.logical_and(cqz + 2 >= 0, cqz - 2 <= gz - 1)
            fone = jnp.full((L,), np.float32(1.0), jnp.float32)
            fzero = jnp.zeros((L,), jnp.float32)
            nr = cqz * 0

            for h in range(2):
                rr = it + h * L
                rx = rr // 5 - 2
                ry = rr % 5 - 2
                cx = jnp.full((L,), cqx, jnp.int32) + rx
                cy = jnp.full((L,), cqy, jnp.int32) + ry
                cxf = lax.convert_element_type(cx, jnp.float32)
                cyf = lax.convert_element_type(cy, jnp.float32)
                dxm = jnp.maximum(
                    jnp.maximum(cxf - uxv, uxv - cxf - fone), fzero)
                dym = jnp.maximum(
                    jnp.maximum(cyf - uyv, uyv - cyf - fone), fzero)
                rxy2 = dxm * dxm + dym * dym
                valid = jnp.logical_and(
                    jnp.logical_and(rr < 25, rxy2 <= tc2),
                    jnp.logical_and(
                        jnp.logical_and(cx >= 0, cx < jnp.full((L,), gx,
                                                            jnp.int32)),
                        jnp.logical_and(cy >= 0, cy < jnp.full((L,), gy,
                                                            jnp.int32))))
                valid = jnp.logical_and(valid, zoks)
                thr = tc2 - rxy2

                def zfail(zc):
                    czf = lax.convert_element_type(zc, jnp.float32)
                    dz = jnp.maximum(
                        jnp.maximum(czf - uzv, uzv - czf - fone), fzero)
                    return lax.convert_element_type(dz * dz > thr,
                                                    jnp.int32)

                zlo_r = jnp.maximum(czv - 2 + zfail(czv - 2)
                                    + zfail(czv - 1), zero)
                zhi_r = jnp.minimum(czv + 2 - zfail(czv + 2)
                                    - zfail(czv + 1),
                                    jnp.full((L,), gz - 1, jnp.int32))
                cxc = jnp.minimum(jnp.maximum(cx, zero),
                                  jnp.full((L,), gx - 1, jnp.int32))
                cyc = jnp.minimum(jnp.maximum(cy, zero),
                                  jnp.full((L,), gy - 1, jnp.int32))
                bc = cxc * jnp.full((L,), sx, jnp.int32) \
                    + cyc * jnp.full((L,), sy, jnp.int32)
                valid = jnp.logical_and(valid, zhi_r >= zlo_r)
                sidx = bc + jnp.maximum(zlo_r, zero)
                eidx = bc + jnp.minimum(zhi_r,
                                        jnp.full((L,), GMAX - 1,
                                                 jnp.int32)) + 1
                s0v = plsc.load_gather(csv, [sidx])
                e0v = plsc.load_gather(csv, [eidx])
                e0v = jnp.where(valid, e0v, s0v)
                mask_r = e0v > s0v
                plsc.store_compressed(rows_s.at[pl.ds(nr, L)], s0v,
                                      mask=mask_r)
                plsc.store_compressed(rows_e.at[pl.ds(nr, L)], e0v,
                                      mask=mask_r)
                nr = nr + plsc.all_reduce_population_count(mask_r)[0]

            def row(r, cnt):
                s0 = _rd(rows_s, r)
                e = _rd(rows_e, r)

                def cond(carry):
                    return carry[0] < e

                def body(carry):
                    j, c = carry
                    rem = e - j
                    lmask = it < jnp.full((L,), rem, jnp.int32)
                    xv = pv[pl.ds(0 * NP + j, L)]
                    yv = pv[pl.ds(1 * NP + j, L)]
                    zv = pv[pl.ds(2 * NP + j, L)]
                    llv = pv[pl.ds(3 * NP + j, L)]
                    mm = (bqx * xv + bqy * yv) + bqz * zv
                    d2 = (qqv + llv) - (np.float32(2.0) * mm)
                    hit = jnp.logical_and(d2 <= r2v, lmask)
                    pc = plsc.all_reduce_population_count(hit)[0]

                    @pl.when(c < MAX_COLLISIONS)
                    def _():
                        posv = lax.convert_element_type(
                            jnp.full((L,), j, jnp.int32) + it, jnp.float32)
                        plsc.store_compressed(
                            stg.at[pl.ds(obase + c, L)], posv, mask=hit)
                    return j + L, c + pc
                _, cnt = lax.while_loop(cond, body, (s0, cnt))
                return cnt
            lax.fori_loop(0, nr, row, np.int32(0))

        def group(qc, _):
            def one_q(qs, _):
                do_query(qc * QG + qs, qs * MAX_COLLISIONS)
                return 0
            lax.fori_loop(0, QG, one_q, 0)
            pltpu.sync_copy(
                stg.at[pl.ds(0, QG * MAX_COLLISIONS)],
                nb_hbm.at[pl.ds(
                    b * (M * MAX_COLLISIONS)
                    + (qbase + qc * QG) * MAX_COLLISIONS,
                    QG * MAX_COLLISIONS)])
            return 0
        lax.fori_loop(0, CHUNK // QG, group, 0)

        for gcp in gathers:
            gcp.wait()

        for ch2 in range(4):
            r0 = ch2 * (DC // 4)

            @pl.loop(r0, r0 + DC // 4)
            def _(i, r0=r0):
                h = (_rd(permL, i) & 1) * C
                for kq in range(C // L):
                    v = plsc.load_gather(
                        drows, [jnp.full((L,), i, jnp.int32),
                                jnp.full((L,), h + kq * L, jnp.int32) + it])
                    bounce[pl.ds((i - r0) * C + kq * L, L)] = v

            pltpu.sync_copy(
                bounce,
                datar_hbm.at[pl.ds(b * (N * C) + (dbase + r0) * C,
                                   (DC // 4) * C)])

    return k(qlocs_T, data2, permi, p_arr, cs, mf, mi)


def kernel(locs, data, qlocs):
    B, N, _ = locs.shape
    M = qlocs.shape[1]
    locs_T = jnp.transpose(locs, (0, 2, 1)).reshape(B * NDIM * N)
    qlocs_T = jnp.transpose(qlocs, (0, 2, 1)).reshape(B * NDIM * M)
    C = data.shape[2]
    data2 = data.reshape(B, (N * C) // 128, 128)
    locsr_T, idxs, permi, p_arr, cs, mf, mi = _phase1(locs_T, B)
    neighbors, data_r = _phase2(qlocs_T, data2, permi, p_arr, cs, mf, mi, C)
    locs_r = jnp.transpose(locsr_T.reshape(B, NDIM, N), (0, 2, 1))
    return (locs_r, data_r.reshape(B, N, C), idxs.reshape(B, N),
            neighbors.reshape(B, M, MAX_COLLISIONS))
